# baseline (speedup 1.0000x reference)
import functools

import jax
import jax.numpy as jnp
from jax import lax
from jax.experimental import pallas as pl
from jax.experimental.pallas import tpu as pltpu
from jax.experimental.pallas import tpu_sc as plsc

N = 10000
E = 320000
D = 128
B = 1024
REG = 1e-05

NC = 2
NS = 16
NW = NC * NS
CHUNK = 64
NCHUNK = E // CHUNK
ZCH = 80
NROWCH = N // ZCH
BPW = B // NW

_mesh = plsc.VectorSubcoreMesh(core_axis_name="c", subcore_axis_name="s")


def _sc_scatter_body(ego_hbm, src_hbm, dst_hbm, att_hbm, out_hbm,
                     acc, ev0, ev1, av0, av1, rows0, rows1,
                     isem0, isem1, gsem0, gsem1, ssem0, ssem1):
    c = lax.axis_index("c")
    s = lax.axis_index("s")
    w = s * NC + c
    n = (NCHUNK - w + NW - 1) // NW
    zlo = (s * NROWCH) // NS
    zhi = ((s + 1) * NROWCH) // NS

    def zero_body(e, _):
        zero = jnp.zeros((16,), jnp.float32)
        for j in range(D // 16):
            rows0[e, pl.ds(j * 16, 16)] = zero
        return 0

    lax.fori_loop(0, ZCH, zero_body, 0)

    def zero_acc_body(k, _):
        row0 = pl.multiple_of(k * ZCH, 8)
        pltpu.sync_copy(rows0.at[pl.ds(0, ZCH)], acc.at[pl.ds(row0, ZCH)])
        return 0

    lax.fori_loop(zlo, zhi, zero_acc_body, 0)
    plsc.subcore_barrier()

    bufs = ((ev0, av0, rows0, isem0, gsem0, ssem0),
            (ev1, av1, rows1, isem1, gsem1, ssem1))

    def issue_idx(g, bb):
        evb, avb = bufs[bb][0], bufs[bb][1]
        cid = w + NW * g
        base = pl.multiple_of(cid * CHUNK, 8)
        abase = pl.multiple_of(cid * CHUNK * 16, 8)
        pltpu.async_copy(src_hbm.at[pl.ds(base, CHUNK)], evb.at[0],
                         bufs[bb][3])
        pltpu.async_copy(dst_hbm.at[pl.ds(base, CHUNK)], evb.at[1],
                         bufs[bb][3])
        pltpu.async_copy(att_hbm.at[pl.ds(abase, CHUNK * 16)], avb,
                         bufs[bb][3])

    def wait_idx(bb):
        evb, avb = bufs[bb][0], bufs[bb][1]
        pltpu.make_async_copy(src_hbm.at[pl.ds(0, CHUNK)], evb.at[0],
                              bufs[bb][3]).wait()
        pltpu.make_async_copy(dst_hbm.at[pl.ds(0, CHUNK)], evb.at[1],
                              bufs[bb][3]).wait()
        pltpu.make_async_copy(att_hbm.at[pl.ds(0, CHUNK * 16)], avb,
                              bufs[bb][3]).wait()

    def wait_scat(bb):
        evb, rowsb = bufs[bb][0], bufs[bb][2]
        pltpu.make_async_copy(rowsb, acc.at[evb.at[1]], bufs[bb][5]).wait()

    def outer(gg, _):
        for b in (0, 1):
            g = 2 * gg + b
            evb, avb, rowsb, isemb, gsemb, ssemb = bufs[b]
            live = g < n

            @pl.when(live)
            def _():
                cid = w + NW * g
                base = pl.multiple_of(cid * CHUNK, 8)
                abase = pl.multiple_of(cid * CHUNK * 16, 8)
                pltpu.sync_copy(src_hbm.at[pl.ds(base, CHUNK)], evb.at[0])
                pltpu.sync_copy(dst_hbm.at[pl.ds(base, CHUNK)], evb.at[1])
                pltpu.sync_copy(att_hbm.at[pl.ds(abase, CHUNK * 16)], avb)
                pltpu.async_copy(ego_hbm.at[evb.at[0]], rowsb, gsemb).wait()

                def scale_body(e, _):
                    ab = avb[pl.ds(e * 16, 16)]
                    for j in range(D // 16):
                        rowsb[e, pl.ds(j * 16, 16)] = (
                            rowsb[e, pl.ds(j * 16, 16)] * ab)
                    return 0

                lax.fori_loop(0, CHUNK, scale_body, 0)
                pltpu.sync_copy(rowsb, acc.at[evb.at[1]], add=True)

        return 0

    lax.fori_loop(0, (NCHUNK // NW + 2) // 2, outer, 0)
    plsc.subcore_barrier()

    def copyout_body(k, _):
        row0 = pl.multiple_of(k * ZCH, 8)
        pltpu.sync_copy(acc.at[pl.ds(row0, ZCH)], rows0.at[pl.ds(0, ZCH)])
        pltpu.sync_copy(rows0.at[pl.ds(0, ZCH)],
                        out_hbm.at[c, pl.ds(row0, ZCH)])
        return 0

    lax.fori_loop(zlo, zhi, copyout_body, 0)


_sc_scatter = pl.kernel(
    _sc_scatter_body,
    out_type=jax.ShapeDtypeStruct((NC, N, D), jnp.float32),
    mesh=_mesh,
    scratch_types=[
        pltpu.VMEM_SHARED((N, D), jnp.float32),
        pltpu.VMEM((2, CHUNK), jnp.int32),
        pltpu.VMEM((2, CHUNK), jnp.int32),
        pltpu.VMEM((CHUNK * 16,), jnp.float32),
        pltpu.VMEM((CHUNK * 16,), jnp.float32),
        pltpu.VMEM((CHUNK, D), jnp.float32),
        pltpu.VMEM((CHUNK, D), jnp.float32),
        pltpu.SemaphoreType.DMA,
        pltpu.SemaphoreType.DMA,
        pltpu.SemaphoreType.DMA,
        pltpu.SemaphoreType.DMA,
        pltpu.SemaphoreType.DMA,
        pltpu.SemaphoreType.DMA,
    ],
)


def _sc_gather_body(t0, t1, t2, ids_hbm, out_hbm, idxv, rowsv, sem):
    c = lax.axis_index("c")
    s = lax.axis_index("s")
    w = s * NC + c
    base = pl.multiple_of(w * BPW, 8)
    for q in range(3):
        pltpu.sync_copy(ids_hbm.at[pl.ds(q * B + base, BPW)], idxv.at[0])
        for t, tab in enumerate((t0, t1, t2)):
            pltpu.async_copy(tab.at[idxv.at[0]], rowsv, sem).wait()
            pltpu.sync_copy(rowsv, out_hbm.at[t, q, pl.ds(base, BPW)])


_sc_gather = pl.kernel(
    _sc_gather_body,
    out_type=jax.ShapeDtypeStruct((3, 3, B, D), jnp.float32),
    mesh=_mesh,
    scratch_types=[
        pltpu.VMEM((1, BPW), jnp.int32),
        pltpu.VMEM((BPW, D), jnp.float32),
        pltpu.SemaphoreType.DMA,
    ],
)


def _tc_layer_body(ego_ref, p_ref, w1_ref, b1_ref, w2_ref, b2_ref,
                   oego_ref, onorm_ref):
    ego = ego_ref[...]
    agg = p_ref[0] + p_ref[1]
    h1 = jnp.dot(ego + agg, w1_ref[...],
                 preferred_element_type=jnp.float32) + b1_ref[...]
    h1 = jnp.where(h1 >= 0, h1, 0.01 * h1)
    h2 = jnp.dot(ego * agg, w2_ref[...],
                 preferred_element_type=jnp.float32) + b2_ref[...]
    h2 = jnp.where(h2 >= 0, h2, 0.01 * h2)
    newego = h1 + h2
    oego_ref[...] = newego
    nrm = jnp.sqrt(jnp.sum(newego * newego, axis=1, keepdims=True)) + 1e-12
    onorm_ref[...] = newego / nrm


_TC_R = 1000


def _tc_layer(ego, parts, w1, b1, w2, b2):
    grid = (N // _TC_R,)
    return pl.pallas_call(
        _tc_layer_body,
        grid=grid,
        in_specs=[
            pl.BlockSpec((_TC_R, D), lambda i: (i, 0)),
            pl.BlockSpec((NC, _TC_R, D), lambda i: (0, i, 0)),
            pl.BlockSpec((D, D), lambda i: (0, 0)),
            pl.BlockSpec((1, D), lambda i: (0, 0)),
            pl.BlockSpec((D, D), lambda i: (0, 0)),
            pl.BlockSpec((1, D), lambda i: (0, 0)),
        ],
        out_specs=[
            pl.BlockSpec((_TC_R, D), lambda i: (i, 0)),
            pl.BlockSpec((_TC_R, D), lambda i: (i, 0)),
        ],
        out_shape=[
            jax.ShapeDtypeStruct((N, D), jnp.float32),
            jax.ShapeDtypeStruct((N, D), jnp.float32),
        ],
    )(ego, parts, w1, b1.reshape(1, D), w2, b2.reshape(1, D))


def _tc_loss_body(g_ref, base_ref, reg_ref):
    sp = jnp.zeros((B, 1), jnp.float32)
    sn = jnp.zeros((B, 1), jnp.float32)
    l2 = jnp.float32(0.0)
    for t in range(3):
        u = g_ref[t, 0]
        p = g_ref[t, 1]
        n = g_ref[t, 2]
        sp = sp + jnp.sum(u * p, axis=1, keepdims=True)
        sn = sn + jnp.sum(u * n, axis=1, keepdims=True)
        l2 = l2 + jnp.sum(u * u) + jnp.sum(p * p) + jnp.sum(n * n)
    x = -(sp - sn)
    softplus = jnp.maximum(x, 0.0) + jnp.log1p(jnp.exp(-jnp.abs(x)))
    base_ref[...] = jnp.sum(softplus).reshape(1, 1)
    reg_ref[...] = (jnp.float32(REG) * 0.5 * l2).reshape(1, 1)


def _tc_loss(gathered):
    return pl.pallas_call(
        _tc_loss_body,
        out_shape=[
            jax.ShapeDtypeStruct((1, 1), jnp.float32),
            jax.ShapeDtypeStruct((1, 1), jnp.float32),
        ],
    )(gathered)


def kernel(entity_table, attention, w1_0, b1_0, w2_0, b2_0, w1_1, b1_1,
           w2_1, b2_1, edge_index, userids, itemids_pos, itemids_neg):
    src = edge_index[0]
    dst = edge_index[1]
    att = jnp.broadcast_to(attention, (E, 16)).reshape(E * 16)

    parts0 = _sc_scatter(entity_table, src, dst, att)
    ego1, norm1 = _tc_layer(entity_table, parts0, w1_0, b1_0, w2_0, b2_0)
    parts1 = _sc_scatter(ego1, src, dst, att)
    _, norm2 = _tc_layer(ego1, parts1, w1_1, b1_1, w2_1, b2_1)

    ids = jnp.concatenate([userids, itemids_pos, itemids_neg], axis=0)
    gathered = _sc_gather(entity_table, norm1, norm2, ids)
    base, reg = _tc_loss(gathered)
    return (base.reshape(()), reg.reshape(()))


# speedup vs baseline: 1.2723x; 1.2723x over previous
import functools

import jax
import jax.numpy as jnp
from jax import lax
from jax.experimental import pallas as pl
from jax.experimental.pallas import tpu as pltpu
from jax.experimental.pallas import tpu_sc as plsc

N = 10000
E = 320000
D = 128
B = 1024
REG = 1e-05

NC = 2
NS = 16
NW = NC * NS
CHUNK = 128
NCHUNK = E // CHUNK
ZCH = 80
NROWCH = N // ZCH
BPW = B // NW

_mesh = plsc.VectorSubcoreMesh(core_axis_name="c", subcore_axis_name="s")


def _sc_scatter_body(ego_hbm, src_hbm, dst_hbm, att_hbm, out_hbm,
                     acc, ev, av, rows, gsem):
    c = lax.axis_index("c")
    s = lax.axis_index("s")
    w = s * NC + c
    n = (NCHUNK - w + NW - 1) // NW
    zlo = (s * NROWCH) // NS
    zhi = ((s + 1) * NROWCH) // NS

    def zero_body(e, _):
        zero = jnp.zeros((16,), jnp.float32)
        for j in range(D // 16):
            rows[e, pl.ds(j * 16, 16)] = zero
        return 0

    lax.fori_loop(0, ZCH, zero_body, 0)

    def zero_acc_body(k, _):
        row0 = pl.multiple_of(k * ZCH, 8)
        pltpu.sync_copy(rows.at[pl.ds(0, ZCH)], acc.at[pl.ds(row0, ZCH)])
        return 0

    lax.fori_loop(zlo, zhi, zero_acc_body, 0)
    plsc.subcore_barrier()

    def chunk_body(g, _):
        live = g < n

        @pl.when(live)
        def _():
            cid = w + NW * g
            base = pl.multiple_of(cid * CHUNK, 8)
            abase = pl.multiple_of(cid * CHUNK * 16, 8)
            pltpu.sync_copy(src_hbm.at[pl.ds(base, CHUNK)], ev.at[0])
            pltpu.sync_copy(dst_hbm.at[pl.ds(base, CHUNK)], ev.at[1])
            pltpu.sync_copy(att_hbm.at[pl.ds(abase, CHUNK * 16)], av)
            pltpu.async_copy(ego_hbm.at[ev.at[0]], rows, gsem).wait()

            def scale_body(e, _):
                ab = av[pl.ds(e * 16, 16)]
                for j in range(D // 16):
                    rows[e, pl.ds(j * 16, 16)] = (
                        rows[e, pl.ds(j * 16, 16)] * ab)
                return 0

            lax.fori_loop(0, CHUNK, scale_body, 0)
            pltpu.sync_copy(rows, acc.at[ev.at[1]], add=True)

        return 0

    lax.fori_loop(0, NCHUNK // NW + 1, chunk_body, 0)
    plsc.subcore_barrier()

    def copyout_body(k, _):
        row0 = pl.multiple_of(k * ZCH, 8)
        pltpu.sync_copy(acc.at[pl.ds(row0, ZCH)], rows.at[pl.ds(0, ZCH)])
        pltpu.sync_copy(rows.at[pl.ds(0, ZCH)],
                        out_hbm.at[c, pl.ds(row0, ZCH)])
        return 0

    lax.fori_loop(zlo, zhi, copyout_body, 0)


_sc_scatter = pl.kernel(
    _sc_scatter_body,
    out_type=jax.ShapeDtypeStruct((NC, N, D), jnp.float32),
    mesh=_mesh,
    scratch_types=[
        pltpu.VMEM_SHARED((N, D), jnp.float32),
        pltpu.VMEM((2, CHUNK), jnp.int32),
        pltpu.VMEM((CHUNK * 16,), jnp.float32),
        pltpu.VMEM((CHUNK, D), jnp.float32),
        pltpu.SemaphoreType.DMA,
    ],
)


def _sc_gather_body(t0, t1, t2, ids_hbm, out_hbm, idxv, rowsv, sem):
    c = lax.axis_index("c")
    s = lax.axis_index("s")
    w = s * NC + c
    base = pl.multiple_of(w * BPW, 8)
    for q in range(3):
        pltpu.sync_copy(ids_hbm.at[pl.ds(q * B + base, BPW)], idxv.at[0])
        for t, tab in enumerate((t0, t1, t2)):
            pltpu.async_copy(tab.at[idxv.at[0]], rowsv, sem).wait()
            pltpu.sync_copy(rowsv, out_hbm.at[t, q, pl.ds(base, BPW)])


_sc_gather = pl.kernel(
    _sc_gather_body,
    out_type=jax.ShapeDtypeStruct((3, 3, B, D), jnp.float32),
    mesh=_mesh,
    scratch_types=[
        pltpu.VMEM((1, BPW), jnp.int32),
        pltpu.VMEM((BPW, D), jnp.float32),
        pltpu.SemaphoreType.DMA,
    ],
)


def _tc_layer_body(ego_ref, p_ref, w1_ref, b1_ref, w2_ref, b2_ref,
                   oego_ref, onorm_ref):
    ego = ego_ref[...]
    agg = p_ref[0] + p_ref[1]
    h1 = jnp.dot(ego + agg, w1_ref[...],
                 preferred_element_type=jnp.float32) + b1_ref[...]
    h1 = jnp.where(h1 >= 0, h1, 0.01 * h1)
    h2 = jnp.dot(ego * agg, w2_ref[...],
                 preferred_element_type=jnp.float32) + b2_ref[...]
    h2 = jnp.where(h2 >= 0, h2, 0.01 * h2)
    newego = h1 + h2
    oego_ref[...] = newego
    nrm = jnp.sqrt(jnp.sum(newego * newego, axis=1, keepdims=True)) + 1e-12
    onorm_ref[...] = newego / nrm


_TC_R = 1000


def _tc_layer(ego, parts, w1, b1, w2, b2):
    grid = (N // _TC_R,)
    return pl.pallas_call(
        _tc_layer_body,
        grid=grid,
        in_specs=[
            pl.BlockSpec((_TC_R, D), lambda i: (i, 0)),
            pl.BlockSpec((NC, _TC_R, D), lambda i: (0, i, 0)),
            pl.BlockSpec((D, D), lambda i: (0, 0)),
            pl.BlockSpec((1, D), lambda i: (0, 0)),
            pl.BlockSpec((D, D), lambda i: (0, 0)),
            pl.BlockSpec((1, D), lambda i: (0, 0)),
        ],
        out_specs=[
            pl.BlockSpec((_TC_R, D), lambda i: (i, 0)),
            pl.BlockSpec((_TC_R, D), lambda i: (i, 0)),
        ],
        out_shape=[
            jax.ShapeDtypeStruct((N, D), jnp.float32),
            jax.ShapeDtypeStruct((N, D), jnp.float32),
        ],
    )(ego, parts, w1, b1.reshape(1, D), w2, b2.reshape(1, D))


def _tc_loss_body(g_ref, base_ref, reg_ref):
    sp = jnp.zeros((B, 1), jnp.float32)
    sn = jnp.zeros((B, 1), jnp.float32)
    l2 = jnp.float32(0.0)
    for t in range(3):
        u = g_ref[t, 0]
        p = g_ref[t, 1]
        n = g_ref[t, 2]
        sp = sp + jnp.sum(u * p, axis=1, keepdims=True)
        sn = sn + jnp.sum(u * n, axis=1, keepdims=True)
        l2 = l2 + jnp.sum(u * u) + jnp.sum(p * p) + jnp.sum(n * n)
    x = -(sp - sn)
    softplus = jnp.maximum(x, 0.0) + jnp.log1p(jnp.exp(-jnp.abs(x)))
    base_ref[...] = jnp.sum(softplus).reshape(1, 1)
    reg_ref[...] = (jnp.float32(REG) * 0.5 * l2).reshape(1, 1)


def _tc_loss(gathered):
    return pl.pallas_call(
        _tc_loss_body,
        out_shape=[
            jax.ShapeDtypeStruct((1, 1), jnp.float32),
            jax.ShapeDtypeStruct((1, 1), jnp.float32),
        ],
    )(gathered)


def kernel(entity_table, attention, w1_0, b1_0, w2_0, b2_0, w1_1, b1_1,
           w2_1, b2_1, edge_index, userids, itemids_pos, itemids_neg):
    src = edge_index[0]
    dst = edge_index[1]
    att = jnp.broadcast_to(attention, (E, 16)).reshape(E * 16)

    parts0 = _sc_scatter(entity_table, src, dst, att)
    ego1, norm1 = _tc_layer(entity_table, parts0, w1_0, b1_0, w2_0, b2_0)
    parts1 = _sc_scatter(ego1, src, dst, att)
    _, norm2 = _tc_layer(ego1, parts1, w1_1, b1_1, w2_1, b2_1)

    ids = jnp.concatenate([userids, itemids_pos, itemids_neg], axis=0)
    gathered = _sc_gather(entity_table, norm1, norm2, ids)
    base, reg = _tc_loss(gathered)
    return (base.reshape(()), reg.reshape(()))


# speedup vs baseline: 1.8656x; 1.4662x over previous
import functools

import jax
import jax.numpy as jnp
from jax import lax
from jax.experimental import pallas as pl
from jax.experimental.pallas import tpu as pltpu
from jax.experimental.pallas import tpu_sc as plsc

N = 10000
E = 320000
D = 128
B = 1024
REG = 1e-05

NC = 2
NS = 16
NW = NC * NS
CHUNK = 128
NCHUNK = E // CHUNK
ZCH = 80
NROWCH = N // ZCH
BPW = B // NW

_mesh = plsc.VectorSubcoreMesh(core_axis_name="c", subcore_axis_name="s")


def _sc_scatter_body(ego_hbm, src_hbm, dst_hbm, att_hbm, out_hbm,
                     acc, ev0, ev1, av0, av1, rows0, rows1,
                     isem0, isem1, gsem0, gsem1, ssem0, ssem1):
    c = lax.axis_index("c")
    s = lax.axis_index("s")
    w = s * NC + c
    n = (NCHUNK - w + NW - 1) // NW
    zlo = (s * NROWCH) // NS
    zhi = ((s + 1) * NROWCH) // NS

    def zero_body(e, _):
        zero = jnp.zeros((16,), jnp.float32)
        for j in range(D // 16):
            rows0[e, pl.ds(j * 16, 16)] = zero
        return 0

    lax.fori_loop(0, ZCH, zero_body, 0)

    def zero_acc_body(k, _):
        row0 = pl.multiple_of(k * ZCH, 8)
        pltpu.sync_copy(rows0.at[pl.ds(0, ZCH)], acc.at[pl.ds(row0, ZCH)])
        return 0

    lax.fori_loop(zlo, zhi, zero_acc_body, 0)
    plsc.subcore_barrier()

    bufs = ((ev0, av0, rows0, isem0, gsem0, ssem0),
            (ev1, av1, rows1, isem1, gsem1, ssem1))

    def issue_idx(g, bb):
        evb, avb, isemb = bb[0], bb[1], bb[3]
        cid = w + NW * g
        base = pl.multiple_of(cid * CHUNK, 8)
        abase = pl.multiple_of(cid * CHUNK * 16, 8)
        pltpu.async_copy(src_hbm.at[pl.ds(base, CHUNK)], evb.at[0], isemb)
        pltpu.async_copy(dst_hbm.at[pl.ds(base, CHUNK)], evb.at[1], isemb)
        pltpu.async_copy(att_hbm.at[pl.ds(abase, CHUNK * 16)], avb, isemb)

    def wait_idx(bb):
        evb, avb, isemb = bb[0], bb[1], bb[3]
        pltpu.make_async_copy(src_hbm.at[pl.ds(0, CHUNK)], evb.at[0],
                              isemb).wait()
        pltpu.make_async_copy(dst_hbm.at[pl.ds(0, CHUNK)], evb.at[1],
                              isemb).wait()
        pltpu.make_async_copy(att_hbm.at[pl.ds(0, CHUNK * 16)], avb,
                              isemb).wait()

    def issue_gather(bb):
        evb, rowsb, gsemb = bb[0], bb[2], bb[4]
        pltpu.async_copy(ego_hbm.at[evb.at[0]], rowsb, gsemb)

    def wait_gather(bb):
        evb, rowsb, gsemb = bb[0], bb[2], bb[4]
        pltpu.make_async_copy(ego_hbm.at[evb.at[0]], rowsb, gsemb).wait()

    def issue_scat(bb):
        evb, rowsb, ssemb = bb[0], bb[2], bb[5]
        pltpu.async_copy(rowsb, acc.at[evb.at[1]], ssemb, add=True)

    def wait_scat(bb):
        evb, rowsb, ssemb = bb[0], bb[2], bb[5]
        pltpu.make_async_copy(rowsb, acc.at[evb.at[1]], ssemb).wait()

    def scale(bb):
        avb, rowsb = bb[1], bb[2]

        def scale_body(e, _):
            ab = avb[pl.ds(e * 16, 16)]
            for j in range(D // 16):
                rowsb[e, pl.ds(j * 16, 16)] = (
                    rowsb[e, pl.ds(j * 16, 16)] * ab)
            return 0

        lax.fori_loop(0, CHUNK, scale_body, 0)

    issue_idx(0, bufs[0])
    wait_idx(bufs[0])
    issue_gather(bufs[0])

    def outer(gg, _):
        for b2 in (0, 1):
            g = 2 * gg + b2
            cur = bufs[b2]
            nxt = bufs[1 - b2]

            @pl.when(g < n)
            def _():
                @pl.when(g + 1 < n)
                def _():
                    @pl.when(g >= 1)
                    def _():
                        wait_scat(nxt)

                    issue_idx(g + 1, nxt)

                wait_gather(cur)

                @pl.when(g + 1 < n)
                def _():
                    wait_idx(nxt)
                    issue_gather(nxt)

                scale(cur)
                issue_scat(cur)

        return 0

    lax.fori_loop(0, (NCHUNK // NW + 2) // 2, outer, 0)
    wait_scat(bufs[0])
    wait_scat(bufs[1])
    plsc.subcore_barrier()

    def copyout_body(k, _):
        row0 = pl.multiple_of(k * ZCH, 8)
        pltpu.sync_copy(acc.at[pl.ds(row0, ZCH)], rows0.at[pl.ds(0, ZCH)])
        pltpu.sync_copy(rows0.at[pl.ds(0, ZCH)],
                        out_hbm.at[c, pl.ds(row0, ZCH)])
        return 0

    lax.fori_loop(zlo, zhi, copyout_body, 0)


_sc_scatter = pl.kernel(
    _sc_scatter_body,
    out_type=jax.ShapeDtypeStruct((NC, N, D), jnp.float32),
    mesh=_mesh,
    scratch_types=[
        pltpu.VMEM_SHARED((N, D), jnp.float32),
        pltpu.VMEM((2, CHUNK), jnp.int32),
        pltpu.VMEM((2, CHUNK), jnp.int32),
        pltpu.VMEM((CHUNK * 16,), jnp.float32),
        pltpu.VMEM((CHUNK * 16,), jnp.float32),
        pltpu.VMEM((CHUNK, D), jnp.float32),
        pltpu.VMEM((CHUNK, D), jnp.float32),
        pltpu.SemaphoreType.DMA,
        pltpu.SemaphoreType.DMA,
        pltpu.SemaphoreType.DMA,
        pltpu.SemaphoreType.DMA,
        pltpu.SemaphoreType.DMA,
        pltpu.SemaphoreType.DMA,
    ],
)


def _sc_gather_body(t0, t1, t2, ids_hbm, out_hbm, idxv, rowsv, sem):
    c = lax.axis_index("c")
    s = lax.axis_index("s")
    w = s * NC + c
    base = pl.multiple_of(w * BPW, 8)
    for q in range(3):
        pltpu.sync_copy(ids_hbm.at[pl.ds(q * B + base, BPW)], idxv.at[0])
        for t, tab in enumerate((t0, t1, t2)):
            pltpu.async_copy(tab.at[idxv.at[0]], rowsv, sem).wait()
            pltpu.sync_copy(rowsv, out_hbm.at[t, q, pl.ds(base, BPW)])


_sc_gather = pl.kernel(
    _sc_gather_body,
    out_type=jax.ShapeDtypeStruct((3, 3, B, D), jnp.float32),
    mesh=_mesh,
    scratch_types=[
        pltpu.VMEM((1, BPW), jnp.int32),
        pltpu.VMEM((BPW, D), jnp.float32),
        pltpu.SemaphoreType.DMA,
    ],
)


def _tc_layer_body(ego_ref, p_ref, w1_ref, b1_ref, w2_ref, b2_ref,
                   oego_ref, onorm_ref):
    ego = ego_ref[...]
    agg = p_ref[0] + p_ref[1]
    h1 = jnp.dot(ego + agg, w1_ref[...],
                 preferred_element_type=jnp.float32) + b1_ref[...]
    h1 = jnp.where(h1 >= 0, h1, 0.01 * h1)
    h2 = jnp.dot(ego * agg, w2_ref[...],
                 preferred_element_type=jnp.float32) + b2_ref[...]
    h2 = jnp.where(h2 >= 0, h2, 0.01 * h2)
    newego = h1 + h2
    oego_ref[...] = newego
    nrm = jnp.sqrt(jnp.sum(newego * newego, axis=1, keepdims=True)) + 1e-12
    onorm_ref[...] = newego / nrm


_TC_R = 1000


def _tc_layer(ego, parts, w1, b1, w2, b2):
    grid = (N // _TC_R,)
    return pl.pallas_call(
        _tc_layer_body,
        grid=grid,
        in_specs=[
            pl.BlockSpec((_TC_R, D), lambda i: (i, 0)),
            pl.BlockSpec((NC, _TC_R, D), lambda i: (0, i, 0)),
            pl.BlockSpec((D, D), lambda i: (0, 0)),
            pl.BlockSpec((1, D), lambda i: (0, 0)),
            pl.BlockSpec((D, D), lambda i: (0, 0)),
            pl.BlockSpec((1, D), lambda i: (0, 0)),
        ],
        out_specs=[
            pl.BlockSpec((_TC_R, D), lambda i: (i, 0)),
            pl.BlockSpec((_TC_R, D), lambda i: (i, 0)),
        ],
        out_shape=[
            jax.ShapeDtypeStruct((N, D), jnp.float32),
            jax.ShapeDtypeStruct((N, D), jnp.float32),
        ],
    )(ego, parts, w1, b1.reshape(1, D), w2, b2.reshape(1, D))


def _tc_loss_body(g_ref, base_ref, reg_ref):
    sp = jnp.zeros((B, 1), jnp.float32)
    sn = jnp.zeros((B, 1), jnp.float32)
    l2 = jnp.float32(0.0)
    for t in range(3):
        u = g_ref[t, 0]
        p = g_ref[t, 1]
        n = g_ref[t, 2]
        sp = sp + jnp.sum(u * p, axis=1, keepdims=True)
        sn = sn + jnp.sum(u * n, axis=1, keepdims=True)
        l2 = l2 + jnp.sum(u * u) + jnp.sum(p * p) + jnp.sum(n * n)
    x = -(sp - sn)
    softplus = jnp.maximum(x, 0.0) + jnp.log1p(jnp.exp(-jnp.abs(x)))
    base_ref[...] = jnp.sum(softplus).reshape(1, 1)
    reg_ref[...] = (jnp.float32(REG) * 0.5 * l2).reshape(1, 1)


def _tc_loss(gathered):
    return pl.pallas_call(
        _tc_loss_body,
        out_shape=[
            jax.ShapeDtypeStruct((1, 1), jnp.float32),
            jax.ShapeDtypeStruct((1, 1), jnp.float32),
        ],
    )(gathered)


def kernel(entity_table, attention, w1_0, b1_0, w2_0, b2_0, w1_1, b1_1,
           w2_1, b2_1, edge_index, userids, itemids_pos, itemids_neg):
    src = edge_index[0]
    dst = edge_index[1]
    att = jnp.broadcast_to(attention, (E, 16)).reshape(E * 16)

    parts0 = _sc_scatter(entity_table, src, dst, att)
    ego1, norm1 = _tc_layer(entity_table, parts0, w1_0, b1_0, w2_0, b2_0)
    parts1 = _sc_scatter(ego1, src, dst, att)
    _, norm2 = _tc_layer(ego1, parts1, w1_1, b1_1, w2_1, b2_1)

    ids = jnp.concatenate([userids, itemids_pos, itemids_neg], axis=0)
    gathered = _sc_gather(entity_table, norm1, norm2, ids)
    base, reg = _tc_loss(gathered)
    return (base.reshape(()), reg.reshape(()))


# speedup vs baseline: 1.9057x; 1.0215x over previous
import functools

import jax
import jax.numpy as jnp
from jax import lax
from jax.experimental import pallas as pl
from jax.experimental.pallas import tpu as pltpu
from jax.experimental.pallas import tpu_sc as plsc

N = 10000
E = 320000
D = 128
B = 1024
REG = 1e-05

NC = 2
NS = 16
NW = NC * NS
CHUNK = 128
NCHUNK = E // CHUNK
ZCH = 80
NROWCH = N // ZCH
BPW = B // NW

_mesh = plsc.VectorSubcoreMesh(core_axis_name="c", subcore_axis_name="s")


def _sc_scatter_body(ego_hbm, src_hbm, dst_hbm, att_hbm, out_hbm,
                     acc, ev0, ev1, av0, av1, rows0, rows1,
                     isem0, isem1, gsem0, gsem1, ssem0, ssem1):
    c = lax.axis_index("c")
    s = lax.axis_index("s")
    w = s * NC + c
    n = (NCHUNK - w + NW - 1) // NW
    zlo = (s * NROWCH) // NS
    zhi = ((s + 1) * NROWCH) // NS

    def zero_body(e, _):
        zero = jnp.zeros((16,), jnp.float32)
        for j in range(D // 16):
            rows0[e, pl.ds(j * 16, 16)] = zero
        return 0

    lax.fori_loop(0, ZCH, zero_body, 0)

    def zero_acc_body(k, _):
        row0 = pl.multiple_of(k * ZCH, 8)
        pltpu.sync_copy(rows0.at[pl.ds(0, ZCH)], acc.at[pl.ds(row0, ZCH)])
        return 0

    lax.fori_loop(zlo, zhi, zero_acc_body, 0)
    plsc.subcore_barrier()

    bufs = ((ev0, av0, rows0, isem0, gsem0, ssem0),
            (ev1, av1, rows1, isem1, gsem1, ssem1))

    def issue_idx(g, bb):
        evb, avb, isemb = bb[0], bb[1], bb[3]
        cid = w + NW * g
        base = pl.multiple_of(cid * CHUNK, 8)
        abase = pl.multiple_of(cid * CHUNK * 16, 8)
        pltpu.async_copy(src_hbm.at[pl.ds(base, CHUNK)], evb.at[0], isemb)
        pltpu.async_copy(dst_hbm.at[pl.ds(base, CHUNK)], evb.at[1], isemb)
        pltpu.async_copy(att_hbm.at[pl.ds(abase, CHUNK * 16)], avb, isemb)

    def wait_idx(bb):
        evb, avb, isemb = bb[0], bb[1], bb[3]
        pltpu.make_async_copy(src_hbm.at[pl.ds(0, CHUNK)], evb.at[0],
                              isemb).wait()
        pltpu.make_async_copy(dst_hbm.at[pl.ds(0, CHUNK)], evb.at[1],
                              isemb).wait()
        pltpu.make_async_copy(att_hbm.at[pl.ds(0, CHUNK * 16)], avb,
                              isemb).wait()

    def issue_gather(bb):
        evb, rowsb, gsemb = bb[0], bb[2], bb[4]
        pltpu.async_copy(ego_hbm.at[evb.at[0]], rowsb, gsemb)

    def wait_gather(bb):
        evb, rowsb, gsemb = bb[0], bb[2], bb[4]
        pltpu.make_async_copy(ego_hbm.at[evb.at[0]], rowsb, gsemb).wait()

    def issue_scat(bb):
        evb, rowsb, ssemb = bb[0], bb[2], bb[5]
        pltpu.async_copy(rowsb, acc.at[evb.at[1]], ssemb, add=True)

    def wait_scat(bb):
        evb, rowsb, ssemb = bb[0], bb[2], bb[5]
        pltpu.make_async_copy(rowsb, acc.at[evb.at[1]], ssemb).wait()

    def scale(bb):
        avb, rowsb = bb[1], bb[2]

        def scale_body(e2, _):
            for u in range(2):
                e = e2 * 2 + u
                ab = avb[pl.ds(e * 16, 16)]
                for j in range(D // 16):
                    rowsb[e, pl.ds(j * 16, 16)] = (
                        rowsb[e, pl.ds(j * 16, 16)] * ab)
            return 0

        lax.fori_loop(0, CHUNK // 2, scale_body, 0)

    issue_idx(0, bufs[0])
    wait_idx(bufs[0])
    issue_gather(bufs[0])

    def outer(gg, _):
        for b2 in (0, 1):
            g = 2 * gg + b2
            cur = bufs[b2]
            nxt = bufs[1 - b2]

            @pl.when(g < n)
            def _():
                @pl.when(g + 1 < n)
                def _():
                    @pl.when(g >= 1)
                    def _():
                        wait_scat(nxt)

                    issue_idx(g + 1, nxt)

                wait_gather(cur)

                @pl.when(g + 1 < n)
                def _():
                    wait_idx(nxt)
                    issue_gather(nxt)

                scale(cur)
                issue_scat(cur)

        return 0

    lax.fori_loop(0, (NCHUNK // NW + 2) // 2, outer, 0)
    wait_scat(bufs[0])
    wait_scat(bufs[1])
    plsc.subcore_barrier()

    def copyout_body(k, _):
        row0 = pl.multiple_of(k * ZCH, 8)
        pltpu.sync_copy(acc.at[pl.ds(row0, ZCH)], rows0.at[pl.ds(0, ZCH)])
        pltpu.sync_copy(rows0.at[pl.ds(0, ZCH)],
                        out_hbm.at[c, pl.ds(row0, ZCH)])
        return 0

    lax.fori_loop(zlo, zhi, copyout_body, 0)


_sc_scatter = pl.kernel(
    _sc_scatter_body,
    out_type=jax.ShapeDtypeStruct((NC, N, D), jnp.float32),
    mesh=_mesh,
    scratch_types=[
        pltpu.VMEM_SHARED((N, D), jnp.float32),
        pltpu.VMEM((2, CHUNK), jnp.int32),
        pltpu.VMEM((2, CHUNK), jnp.int32),
        pltpu.VMEM((CHUNK * 16,), jnp.float32),
        pltpu.VMEM((CHUNK * 16,), jnp.float32),
        pltpu.VMEM((CHUNK, D), jnp.float32),
        pltpu.VMEM((CHUNK, D), jnp.float32),
        pltpu.SemaphoreType.DMA,
        pltpu.SemaphoreType.DMA,
        pltpu.SemaphoreType.DMA,
        pltpu.SemaphoreType.DMA,
        pltpu.SemaphoreType.DMA,
        pltpu.SemaphoreType.DMA,
    ],
)


def _sc_gather_body(t0, t1, t2, ids_hbm, out_hbm, idxv, rowsv, sem):
    c = lax.axis_index("c")
    s = lax.axis_index("s")
    w = s * NC + c
    base = pl.multiple_of(w * BPW, 8)
    for q in range(3):
        pltpu.sync_copy(ids_hbm.at[pl.ds(q * B + base, BPW)], idxv.at[0])
        for t, tab in enumerate((t0, t1, t2)):
            pltpu.async_copy(tab.at[idxv.at[0]], rowsv, sem).wait()
            pltpu.sync_copy(rowsv, out_hbm.at[t, q, pl.ds(base, BPW)])


_sc_gather = pl.kernel(
    _sc_gather_body,
    out_type=jax.ShapeDtypeStruct((3, 3, B, D), jnp.float32),
    mesh=_mesh,
    scratch_types=[
        pltpu.VMEM((1, BPW), jnp.int32),
        pltpu.VMEM((BPW, D), jnp.float32),
        pltpu.SemaphoreType.DMA,
    ],
)


def _tc_layer_body(ego_ref, p_ref, w1_ref, b1_ref, w2_ref, b2_ref,
                   oego_ref, onorm_ref):
    ego = ego_ref[...]
    agg = p_ref[0] + p_ref[1]
    h1 = jnp.dot(ego + agg, w1_ref[...],
                 preferred_element_type=jnp.float32) + b1_ref[...]
    h1 = jnp.where(h1 >= 0, h1, 0.01 * h1)
    h2 = jnp.dot(ego * agg, w2_ref[...],
                 preferred_element_type=jnp.float32) + b2_ref[...]
    h2 = jnp.where(h2 >= 0, h2, 0.01 * h2)
    newego = h1 + h2
    oego_ref[...] = newego
    nrm = jnp.sqrt(jnp.sum(newego * newego, axis=1, keepdims=True)) + 1e-12
    onorm_ref[...] = newego / nrm


_TC_R = 1000


def _tc_layer(ego, parts, w1, b1, w2, b2):
    grid = (N // _TC_R,)
    return pl.pallas_call(
        _tc_layer_body,
        grid=grid,
        in_specs=[
            pl.BlockSpec((_TC_R, D), lambda i: (i, 0)),
            pl.BlockSpec((NC, _TC_R, D), lambda i: (0, i, 0)),
            pl.BlockSpec((D, D), lambda i: (0, 0)),
            pl.BlockSpec((1, D), lambda i: (0, 0)),
            pl.BlockSpec((D, D), lambda i: (0, 0)),
            pl.BlockSpec((1, D), lambda i: (0, 0)),
        ],
        out_specs=[
            pl.BlockSpec((_TC_R, D), lambda i: (i, 0)),
            pl.BlockSpec((_TC_R, D), lambda i: (i, 0)),
        ],
        out_shape=[
            jax.ShapeDtypeStruct((N, D), jnp.float32),
            jax.ShapeDtypeStruct((N, D), jnp.float32),
        ],
    )(ego, parts, w1, b1.reshape(1, D), w2, b2.reshape(1, D))


def _tc_loss_body(g_ref, base_ref, reg_ref):
    sp = jnp.zeros((B, 1), jnp.float32)
    sn = jnp.zeros((B, 1), jnp.float32)
    l2 = jnp.float32(0.0)
    for t in range(3):
        u = g_ref[t, 0]
        p = g_ref[t, 1]
        n = g_ref[t, 2]
        sp = sp + jnp.sum(u * p, axis=1, keepdims=True)
        sn = sn + jnp.sum(u * n, axis=1, keepdims=True)
        l2 = l2 + jnp.sum(u * u) + jnp.sum(p * p) + jnp.sum(n * n)
    x = -(sp - sn)
    softplus = jnp.maximum(x, 0.0) + jnp.log1p(jnp.exp(-jnp.abs(x)))
    base_ref[...] = jnp.sum(softplus).reshape(1, 1)
    reg_ref[...] = (jnp.float32(REG) * 0.5 * l2).reshape(1, 1)


def _tc_loss(gathered):
    return pl.pallas_call(
        _tc_loss_body,
        out_shape=[
            jax.ShapeDtypeStruct((1, 1), jnp.float32),
            jax.ShapeDtypeStruct((1, 1), jnp.float32),
        ],
    )(gathered)


def kernel(entity_table, attention, w1_0, b1_0, w2_0, b2_0, w1_1, b1_1,
           w2_1, b2_1, edge_index, userids, itemids_pos, itemids_neg):
    src = edge_index[0]
    dst = edge_index[1]
    att = jnp.broadcast_to(attention, (E, 16)).reshape(E * 16)

    parts0 = _sc_scatter(entity_table, src, dst, att)
    ego1, norm1 = _tc_layer(entity_table, parts0, w1_0, b1_0, w2_0, b2_0)
    parts1 = _sc_scatter(ego1, src, dst, att)
    _, norm2 = _tc_layer(ego1, parts1, w1_1, b1_1, w2_1, b2_1)

    ids = jnp.concatenate([userids, itemids_pos, itemids_neg], axis=0)
    gathered = _sc_gather(entity_table, norm1, norm2, ids)
    base, reg = _tc_loss(gathered)
    return (base.reshape(()), reg.reshape(()))


# speedup vs baseline: 2.4657x; 1.2939x over previous
import functools

import jax
import jax.numpy as jnp
from jax import lax
from jax.experimental import pallas as pl
from jax.experimental.pallas import tpu as pltpu
from jax.experimental.pallas import tpu_sc as plsc

N = 10000
E = 320000
D = 128
B = 1024
REG = 1e-05

NC = 2
NS = 16
NW = NC * NS
CHUNK = 64
NCHUNK = E // CHUNK
ZCH = 40
NROWCH = N // ZCH
BPW = B // NW

_mesh = plsc.VectorSubcoreMesh(core_axis_name="c", subcore_axis_name="s")


def _sc_scatter_body(ego_hbm, src_hbm, dst_hbm, att_hbm, out_hbm,
                     acc,
                     ev0, ev1, ev2, ev3, ev4,
                     av0, av1, av2, av3, av4,
                     rows0, rows1, rows2, rows3, rows4,
                     isem0, isem1, isem2, isem3, isem4,
                     gsem0, gsem1, gsem2, gsem3, gsem4,
                     ssem0, ssem1, ssem2, ssem3, ssem4):
    c = lax.axis_index("c")
    s = lax.axis_index("s")
    w = s * NC + c
    n = (NCHUNK - w + NW - 1) // NW
    zlo = (s * NROWCH) // NS
    zhi = ((s + 1) * NROWCH) // NS

    def zero_body(e, _):
        zero = jnp.zeros((16,), jnp.float32)
        for j in range(D // 16):
            rows0[e, pl.ds(j * 16, 16)] = zero
        return 0

    lax.fori_loop(0, ZCH, zero_body, 0)

    def zero_acc_body(k, _):
        row0 = pl.multiple_of(k * ZCH, 8)
        pltpu.sync_copy(rows0.at[pl.ds(0, ZCH)], acc.at[pl.ds(row0, ZCH)])
        return 0

    lax.fori_loop(zlo, zhi, zero_acc_body, 0)
    plsc.subcore_barrier()

    bufs = ((ev0, av0, rows0, isem0, gsem0, ssem0),
            (ev1, av1, rows1, isem1, gsem1, ssem1),
            (ev2, av2, rows2, isem2, gsem2, ssem2),
            (ev3, av3, rows3, isem3, gsem3, ssem3),
            (ev4, av4, rows4, isem4, gsem4, ssem4))

    def issue_idx(g, bb):
        evb, avb, isemb = bb[0], bb[1], bb[3]
        cid = w + NW * g
        base = pl.multiple_of(cid * CHUNK, 8)
        abase = pl.multiple_of(cid * CHUNK * 16, 8)
        pltpu.async_copy(src_hbm.at[pl.ds(base, CHUNK)], evb.at[0], isemb)
        pltpu.async_copy(dst_hbm.at[pl.ds(base, CHUNK)], evb.at[1], isemb)
        pltpu.async_copy(att_hbm.at[pl.ds(abase, CHUNK * 16)], avb, isemb)

    def wait_idx(bb):
        evb, avb, isemb = bb[0], bb[1], bb[3]
        pltpu.make_async_copy(src_hbm.at[pl.ds(0, CHUNK)], evb.at[0],
                              isemb).wait()
        pltpu.make_async_copy(dst_hbm.at[pl.ds(0, CHUNK)], evb.at[1],
                              isemb).wait()
        pltpu.make_async_copy(att_hbm.at[pl.ds(0, CHUNK * 16)], avb,
                              isemb).wait()

    def issue_gather(bb):
        evb, rowsb, gsemb = bb[0], bb[2], bb[4]
        pltpu.async_copy(ego_hbm.at[evb.at[0]], rowsb, gsemb)

    def wait_gather(bb):
        evb, rowsb, gsemb = bb[0], bb[2], bb[4]
        pltpu.make_async_copy(ego_hbm.at[evb.at[0]], rowsb, gsemb).wait()

    def issue_scat(bb):
        evb, rowsb, ssemb = bb[0], bb[2], bb[5]
        pltpu.async_copy(rowsb, acc.at[evb.at[1]], ssemb, add=True)

    def wait_scat(bb):
        evb, rowsb, ssemb = bb[0], bb[2], bb[5]
        pltpu.make_async_copy(rowsb, acc.at[evb.at[1]], ssemb).wait()

    def scale(bb):
        avb, rowsb = bb[1], bb[2]

        def scale_body(e2, _):
            for u in range(2):
                e = e2 * 2 + u
                ab = avb[pl.ds(e * 16, 16)]
                for j in range(D // 16):
                    rowsb[e, pl.ds(j * 16, 16)] = (
                        rowsb[e, pl.ds(j * 16, 16)] * ab)
            return 0

        lax.fori_loop(0, CHUNK // 2, scale_body, 0)

    issue_idx(0, bufs[0])
    issue_idx(1, bufs[1])
    issue_idx(2, bufs[2])
    wait_idx(bufs[0])
    issue_gather(bufs[0])
    wait_idx(bufs[1])
    issue_gather(bufs[1])

    K = len(bufs)

    def outer(gg, _):
        for b5 in range(K):
            g = K * gg + b5
            cur = bufs[b5]

            @pl.when(g < n)
            def _():
                @pl.when(g + 3 < n)
                def _():
                    nb = bufs[(b5 + 3) % K]

                    @pl.when(g >= 2)
                    def _():
                        wait_scat(nb)

                    issue_idx(g + 3, nb)

                @pl.when(g + 2 < n)
                def _():
                    gb = bufs[(b5 + 2) % K]
                    wait_idx(gb)
                    issue_gather(gb)

                wait_gather(cur)
                scale(cur)
                issue_scat(cur)

        return 0

    lax.fori_loop(0, (NCHUNK // NW + 1 + K - 1) // K, outer, 0)
    for bb in bufs:
        wait_scat(bb)
    plsc.subcore_barrier()

    def copyout_body(k, _):
        row0 = pl.multiple_of(k * ZCH, 8)
        pltpu.sync_copy(acc.at[pl.ds(row0, ZCH)], rows0.at[pl.ds(0, ZCH)])
        pltpu.sync_copy(rows0.at[pl.ds(0, ZCH)],
                        out_hbm.at[c, pl.ds(row0, ZCH)])
        return 0

    lax.fori_loop(zlo, zhi, copyout_body, 0)


_sc_scatter = pl.kernel(
    _sc_scatter_body,
    out_type=jax.ShapeDtypeStruct((NC, N, D), jnp.float32),
    mesh=_mesh,
    scratch_types=(
        [pltpu.VMEM_SHARED((N, D), jnp.float32)]
        + [pltpu.VMEM((2, CHUNK), jnp.int32)] * 5
        + [pltpu.VMEM((CHUNK * 16,), jnp.float32)] * 5
        + [pltpu.VMEM((CHUNK, D), jnp.float32)] * 5
        + [pltpu.SemaphoreType.DMA] * 15
    ),
)


def _sc_gather_body(t0, t1, t2, ids_hbm, out_hbm, idxv, rowsv, sem):
    c = lax.axis_index("c")
    s = lax.axis_index("s")
    w = s * NC + c
    base = pl.multiple_of(w * BPW, 8)
    for q in range(3):
        pltpu.sync_copy(ids_hbm.at[pl.ds(q * B + base, BPW)], idxv.at[0])
        for t, tab in enumerate((t0, t1, t2)):
            pltpu.async_copy(tab.at[idxv.at[0]], rowsv, sem).wait()
            pltpu.sync_copy(rowsv, out_hbm.at[t, q, pl.ds(base, BPW)])


_sc_gather = pl.kernel(
    _sc_gather_body,
    out_type=jax.ShapeDtypeStruct((3, 3, B, D), jnp.float32),
    mesh=_mesh,
    scratch_types=[
        pltpu.VMEM((1, BPW), jnp.int32),
        pltpu.VMEM((BPW, D), jnp.float32),
        pltpu.SemaphoreType.DMA,
    ],
)


def _tc_layer_body(ego_ref, p_ref, w1_ref, b1_ref, w2_ref, b2_ref,
                   oego_ref, onorm_ref):
    ego = ego_ref[...]
    agg = p_ref[0] + p_ref[1]
    h1 = jnp.dot(ego + agg, w1_ref[...],
                 preferred_element_type=jnp.float32) + b1_ref[...]
    h1 = jnp.where(h1 >= 0, h1, 0.01 * h1)
    h2 = jnp.dot(ego * agg, w2_ref[...],
                 preferred_element_type=jnp.float32) + b2_ref[...]
    h2 = jnp.where(h2 >= 0, h2, 0.01 * h2)
    newego = h1 + h2
    oego_ref[...] = newego
    nrm = jnp.sqrt(jnp.sum(newego * newego, axis=1, keepdims=True)) + 1e-12
    onorm_ref[...] = newego / nrm


_TC_R = 1000


def _tc_layer(ego, parts, w1, b1, w2, b2):
    grid = (N // _TC_R,)
    return pl.pallas_call(
        _tc_layer_body,
        grid=grid,
        in_specs=[
            pl.BlockSpec((_TC_R, D), lambda i: (i, 0)),
            pl.BlockSpec((NC, _TC_R, D), lambda i: (0, i, 0)),
            pl.BlockSpec((D, D), lambda i: (0, 0)),
            pl.BlockSpec((1, D), lambda i: (0, 0)),
            pl.BlockSpec((D, D), lambda i: (0, 0)),
            pl.BlockSpec((1, D), lambda i: (0, 0)),
        ],
        out_specs=[
            pl.BlockSpec((_TC_R, D), lambda i: (i, 0)),
            pl.BlockSpec((_TC_R, D), lambda i: (i, 0)),
        ],
        out_shape=[
            jax.ShapeDtypeStruct((N, D), jnp.float32),
            jax.ShapeDtypeStruct((N, D), jnp.float32),
        ],
    )(ego, parts, w1, b1.reshape(1, D), w2, b2.reshape(1, D))


def _tc_loss_body(g_ref, base_ref, reg_ref):
    sp = jnp.zeros((B, 1), jnp.float32)
    sn = jnp.zeros((B, 1), jnp.float32)
    l2 = jnp.float32(0.0)
    for t in range(3):
        u = g_ref[t, 0]
        p = g_ref[t, 1]
        n = g_ref[t, 2]
        sp = sp + jnp.sum(u * p, axis=1, keepdims=True)
        sn = sn + jnp.sum(u * n, axis=1, keepdims=True)
        l2 = l2 + jnp.sum(u * u) + jnp.sum(p * p) + jnp.sum(n * n)
    x = -(sp - sn)
    softplus = jnp.maximum(x, 0.0) + jnp.log1p(jnp.exp(-jnp.abs(x)))
    base_ref[...] = jnp.sum(softplus).reshape(1, 1)
    reg_ref[...] = (jnp.float32(REG) * 0.5 * l2).reshape(1, 1)


def _tc_loss(gathered):
    return pl.pallas_call(
        _tc_loss_body,
        out_shape=[
            jax.ShapeDtypeStruct((1, 1), jnp.float32),
            jax.ShapeDtypeStruct((1, 1), jnp.float32),
        ],
    )(gathered)


def kernel(entity_table, attention, w1_0, b1_0, w2_0, b2_0, w1_1, b1_1,
           w2_1, b2_1, edge_index, userids, itemids_pos, itemids_neg):
    src = edge_index[0]
    dst = edge_index[1]
    att = jnp.broadcast_to(attention, (E, 16)).reshape(E * 16)

    parts0 = _sc_scatter(entity_table, src, dst, att)
    ego1, norm1 = _tc_layer(entity_table, parts0, w1_0, b1_0, w2_0, b2_0)
    parts1 = _sc_scatter(ego1, src, dst, att)
    _, norm2 = _tc_layer(ego1, parts1, w1_1, b1_1, w2_1, b2_1)

    ids = jnp.concatenate([userids, itemids_pos, itemids_neg], axis=0)
    gathered = _sc_gather(entity_table, norm1, norm2, ids)
    base, reg = _tc_loss(gathered)
    return (base.reshape(()), reg.reshape(()))


# speedup vs baseline: 4.3999x; 1.7845x over previous
import functools

import jax
import jax.numpy as jnp
from jax import lax
from jax.experimental import pallas as pl
from jax.experimental.pallas import tpu as pltpu
from jax.experimental.pallas import tpu_sc as plsc

N = 10000
E = 320000
D = 128
B = 1024
REG = 1e-05

NC = 2
NS = 16
NW = NC * NS
CHUNK = 64
NCHUNK = E // CHUNK
ZCH = 40
NROWCH = N // ZCH
BPW = B // NW

_mesh = plsc.VectorSubcoreMesh(core_axis_name="c", subcore_axis_name="s")


def _sc_scatter_body(ego_hbm, src_hbm, dst_hbm, att_hbm, out_hbm,
                     acc,
                     ev0, ev1, ev2, ev3, ev4,
                     av0, av1, av2, av3, av4,
                     rows0, rows1, rows2, rows3, rows4,
                     isem0, isem1, isem2, isem3, isem4,
                     gsem0, gsem1, gsem2, gsem3, gsem4,
                     ssem0, ssem1, ssem2, ssem3, ssem4):
    c = lax.axis_index("c")
    s = lax.axis_index("s")
    w = s * NC + c
    n = (NCHUNK - w + NW - 1) // NW
    zlo = (s * NROWCH) // NS
    zhi = ((s + 1) * NROWCH) // NS

    def zero_body(e, _):
        zero = jnp.zeros((16,), jnp.float32)
        for j in range(D // 16):
            rows0[e, pl.ds(j * 16, 16)] = zero
        return 0

    lax.fori_loop(0, ZCH, zero_body, 0)

    def zero_acc_body(k, _):
        row0 = pl.multiple_of(k * ZCH, 8)
        pltpu.sync_copy(rows0.at[pl.ds(0, ZCH)], acc.at[pl.ds(row0, ZCH)])
        return 0

    lax.fori_loop(zlo, zhi, zero_acc_body, 0)
    plsc.subcore_barrier()

    bufs = ((ev0, av0, rows0, isem0, gsem0, ssem0),
            (ev1, av1, rows1, isem1, gsem1, ssem1),
            (ev2, av2, rows2, isem2, gsem2, ssem2),
            (ev3, av3, rows3, isem3, gsem3, ssem3),
            (ev4, av4, rows4, isem4, gsem4, ssem4))

    def issue_idx(g, bb):
        evb, avb, isemb = bb[0], bb[1], bb[3]
        cid = w + NW * g
        base = pl.multiple_of(cid * CHUNK, 8)
        pltpu.async_copy(src_hbm.at[pl.ds(base, CHUNK)], evb.at[0], isemb)
        pltpu.async_copy(dst_hbm.at[pl.ds(base, CHUNK)], evb.at[1], isemb)
        pltpu.async_copy(att_hbm.at[pl.ds(base, CHUNK)], avb, isemb)

    def wait_idx(bb):
        evb, avb, isemb = bb[0], bb[1], bb[3]
        pltpu.make_async_copy(src_hbm.at[pl.ds(0, CHUNK)], evb.at[0],
                              isemb).wait()
        pltpu.make_async_copy(dst_hbm.at[pl.ds(0, CHUNK)], evb.at[1],
                              isemb).wait()
        pltpu.make_async_copy(att_hbm.at[pl.ds(0, CHUNK)], avb,
                              isemb).wait()

    def issue_gather(bb):
        evb, rowsb, gsemb = bb[0], bb[2], bb[4]
        pltpu.async_copy(ego_hbm.at[evb.at[0]], rowsb, gsemb)

    def wait_gather(bb):
        evb, rowsb, gsemb = bb[0], bb[2], bb[4]
        pltpu.make_async_copy(ego_hbm.at[evb.at[0]], rowsb, gsemb).wait()

    def issue_scat(bb):
        evb, rowsb, ssemb = bb[0], bb[2], bb[5]
        pltpu.async_copy(rowsb, acc.at[evb.at[1]], ssemb, add=True)

    def wait_scat(bb):
        evb, rowsb, ssemb = bb[0], bb[2], bb[5]
        pltpu.make_async_copy(rowsb, acc.at[evb.at[1]], ssemb).wait()

    def scale(bb):
        avb, rowsb = bb[1], bb[2]

        def scale_body(e16, _):
            base16 = e16 * 16
            seg = avb[pl.ds(base16, 16)]
            for u in range(16):
                e = base16 + u
                ab = seg.at[jnp.full((16,), u, jnp.int32)].get(
                    mode="promise_in_bounds")
                for j in range(D // 16):
                    rowsb[e, pl.ds(j * 16, 16)] = (
                        rowsb[e, pl.ds(j * 16, 16)] * ab)
            return 0

        lax.fori_loop(0, CHUNK // 16, scale_body, 0)

    issue_idx(0, bufs[0])
    issue_idx(1, bufs[1])
    issue_idx(2, bufs[2])
    wait_idx(bufs[0])
    issue_gather(bufs[0])
    wait_idx(bufs[1])
    issue_gather(bufs[1])

    K = len(bufs)

    def outer(gg, _):
        for b5 in range(K):
            g = K * gg + b5
            cur = bufs[b5]

            @pl.when(g < n)
            def _():
                @pl.when(g + 3 < n)
                def _():
                    nb = bufs[(b5 + 3) % K]

                    @pl.when(g >= 2)
                    def _():
                        wait_scat(nb)

                    issue_idx(g + 3, nb)

                @pl.when(g + 2 < n)
                def _():
                    gb = bufs[(b5 + 2) % K]
                    wait_idx(gb)
                    issue_gather(gb)

                wait_gather(cur)
                scale(cur)
                issue_scat(cur)

        return 0

    lax.fori_loop(0, (NCHUNK // NW + 1 + K - 1) // K, outer, 0)
    for bb in bufs:
        wait_scat(bb)
    plsc.subcore_barrier()

    def copyout_body(k, _):
        row0 = pl.multiple_of(k * ZCH, 8)
        pltpu.sync_copy(acc.at[pl.ds(row0, ZCH)], rows0.at[pl.ds(0, ZCH)])
        pltpu.sync_copy(rows0.at[pl.ds(0, ZCH)],
                        out_hbm.at[c, pl.ds(row0, ZCH)])
        return 0

    lax.fori_loop(zlo, zhi, copyout_body, 0)


_sc_scatter = pl.kernel(
    _sc_scatter_body,
    out_type=jax.ShapeDtypeStruct((NC, N, D), jnp.float32),
    mesh=_mesh,
    scratch_types=(
        [pltpu.VMEM_SHARED((N, D), jnp.float32)]
        + [pltpu.VMEM((2, CHUNK), jnp.int32)] * 5
        + [pltpu.VMEM((CHUNK,), jnp.float32)] * 5
        + [pltpu.VMEM((CHUNK, D), jnp.float32)] * 5
        + [pltpu.SemaphoreType.DMA] * 15
    ),
)


def _sc_gather_body(t0, t1, t2, uids, pids, nids, out_hbm, idxv, rowsv, sem):
    c = lax.axis_index("c")
    s = lax.axis_index("s")
    w = s * NC + c
    base = pl.multiple_of(w * BPW, 8)
    for q, ids_hbm in enumerate((uids, pids, nids)):
        pltpu.sync_copy(ids_hbm.at[pl.ds(base, BPW)], idxv.at[0])
        for t, tab in enumerate((t0, t1, t2)):
            pltpu.async_copy(tab.at[idxv.at[0]], rowsv, sem).wait()
            pltpu.sync_copy(rowsv, out_hbm.at[t, q, pl.ds(base, BPW)])


_sc_gather = pl.kernel(
    _sc_gather_body,
    out_type=jax.ShapeDtypeStruct((3, 3, B, D), jnp.float32),
    mesh=_mesh,
    scratch_types=[
        pltpu.VMEM((1, BPW), jnp.int32),
        pltpu.VMEM((BPW, D), jnp.float32),
        pltpu.SemaphoreType.DMA,
    ],
)


def _tc_layer_body(ego_ref, p_ref, w1_ref, b1_ref, w2_ref, b2_ref,
                   oego_ref, onorm_ref):
    ego = ego_ref[...]
    agg = p_ref[0] + p_ref[1]
    h1 = jnp.dot(ego + agg, w1_ref[...],
                 preferred_element_type=jnp.float32) + b1_ref[...]
    h1 = jnp.where(h1 >= 0, h1, 0.01 * h1)
    h2 = jnp.dot(ego * agg, w2_ref[...],
                 preferred_element_type=jnp.float32) + b2_ref[...]
    h2 = jnp.where(h2 >= 0, h2, 0.01 * h2)
    newego = h1 + h2
    oego_ref[...] = newego
    nrm = jnp.sqrt(jnp.sum(newego * newego, axis=1, keepdims=True)) + 1e-12
    onorm_ref[...] = newego / nrm


_TC_R = 1000


def _tc_layer(ego, parts, w1, b1, w2, b2):
    grid = (N // _TC_R,)
    return pl.pallas_call(
        _tc_layer_body,
        grid=grid,
        in_specs=[
            pl.BlockSpec((_TC_R, D), lambda i: (i, 0)),
            pl.BlockSpec((NC, _TC_R, D), lambda i: (0, i, 0)),
            pl.BlockSpec((D, D), lambda i: (0, 0)),
            pl.BlockSpec((1, D), lambda i: (0, 0)),
            pl.BlockSpec((D, D), lambda i: (0, 0)),
            pl.BlockSpec((1, D), lambda i: (0, 0)),
        ],
        out_specs=[
            pl.BlockSpec((_TC_R, D), lambda i: (i, 0)),
            pl.BlockSpec((_TC_R, D), lambda i: (i, 0)),
        ],
        out_shape=[
            jax.ShapeDtypeStruct((N, D), jnp.float32),
            jax.ShapeDtypeStruct((N, D), jnp.float32),
        ],
    )(ego, parts, w1, b1.reshape(1, D), w2, b2.reshape(1, D))


def _tc_loss_body(g_ref, base_ref, reg_ref):
    sp = jnp.zeros((B, 1), jnp.float32)
    sn = jnp.zeros((B, 1), jnp.float32)
    l2 = jnp.float32(0.0)
    for t in range(3):
        u = g_ref[t, 0]
        p = g_ref[t, 1]
        n = g_ref[t, 2]
        sp = sp + jnp.sum(u * p, axis=1, keepdims=True)
        sn = sn + jnp.sum(u * n, axis=1, keepdims=True)
        l2 = l2 + jnp.sum(u * u) + jnp.sum(p * p) + jnp.sum(n * n)
    x = -(sp - sn)
    softplus = jnp.maximum(x, 0.0) + jnp.log1p(jnp.exp(-jnp.abs(x)))
    base_ref[...] = jnp.sum(softplus).reshape(1, 1)
    reg_ref[...] = (jnp.float32(REG) * 0.5 * l2).reshape(1, 1)


def _tc_loss(gathered):
    return pl.pallas_call(
        _tc_loss_body,
        out_shape=[
            jax.ShapeDtypeStruct((1, 1), jnp.float32),
            jax.ShapeDtypeStruct((1, 1), jnp.float32),
        ],
    )(gathered)


def kernel(entity_table, attention, w1_0, b1_0, w2_0, b2_0, w1_1, b1_1,
           w2_1, b2_1, edge_index, userids, itemids_pos, itemids_neg):
    src = edge_index[0]
    dst = edge_index[1]
    att = attention.reshape(E)

    parts0 = _sc_scatter(entity_table, src, dst, att)
    ego1, norm1 = _tc_layer(entity_table, parts0, w1_0, b1_0, w2_0, b2_0)
    parts1 = _sc_scatter(ego1, src, dst, att)
    _, norm2 = _tc_layer(ego1, parts1, w1_1, b1_1, w2_1, b2_1)

    gathered = _sc_gather(entity_table, norm1, norm2, userids, itemids_pos,
                          itemids_neg)
    base, reg = _tc_loss(gathered)
    return (base.reshape(()), reg.reshape(()))


# speedup vs baseline: 4.5650x; 1.0375x over previous
import functools

import jax
import jax.numpy as jnp
from jax import lax
from jax.experimental import pallas as pl
from jax.experimental.pallas import tpu as pltpu
from jax.experimental.pallas import tpu_sc as plsc

N = 10000
E = 320000
D = 128
B = 1024
REG = 1e-05

NC = 2
NS = 16
NW = NC * NS
CHUNK = 64
NCHUNK = E // CHUNK
ZCH = 40
NROWCH = N // ZCH
BPW = B // NW

_mesh = plsc.VectorSubcoreMesh(core_axis_name="c", subcore_axis_name="s")


def _sc_scatter_body(ego_hbm, src_hbm, dst_hbm, att_hbm, out_hbm,
                     acc,
                     ev0, ev1, ev2, ev3, ev4,
                     av0, av1, av2, av3, av4,
                     rows0, rows1, rows2, rows3, rows4,
                     isem0, isem1, isem2, isem3, isem4,
                     gsem0, gsem1, gsem2, gsem3, gsem4,
                     ssem0, ssem1, ssem2, ssem3, ssem4):
    c = lax.axis_index("c")
    s = lax.axis_index("s")
    w = s * NC + c
    n = (NCHUNK - w + NW - 1) // NW
    zlo = (s * NROWCH) // NS
    zhi = ((s + 1) * NROWCH) // NS

    nz = zhi - zlo

    bufs = ((ev0, av0, rows0, isem0, gsem0, ssem0),
            (ev1, av1, rows1, isem1, gsem1, ssem1),
            (ev2, av2, rows2, isem2, gsem2, ssem2),
            (ev3, av3, rows3, isem3, gsem3, ssem3),
            (ev4, av4, rows4, isem4, gsem4, ssem4))

    def issue_idx(g, bb):
        evb, avb, isemb = bb[0], bb[1], bb[3]
        cid = w + NW * g
        base = pl.multiple_of(cid * CHUNK, 8)
        pltpu.async_copy(src_hbm.at[pl.ds(base, CHUNK)], evb.at[0], isemb)
        pltpu.async_copy(dst_hbm.at[pl.ds(base, CHUNK)], evb.at[1], isemb)
        pltpu.async_copy(att_hbm.at[pl.ds(base, CHUNK)], avb, isemb)

    def wait_idx(bb):
        evb, avb, isemb = bb[0], bb[1], bb[3]
        pltpu.make_async_copy(src_hbm.at[pl.ds(0, CHUNK)], evb.at[0],
                              isemb).wait()
        pltpu.make_async_copy(dst_hbm.at[pl.ds(0, CHUNK)], evb.at[1],
                              isemb).wait()
        pltpu.make_async_copy(att_hbm.at[pl.ds(0, CHUNK)], avb,
                              isemb).wait()

    def issue_gather(bb):
        evb, rowsb, gsemb = bb[0], bb[2], bb[4]
        pltpu.async_copy(ego_hbm.at[evb.at[0]], rowsb, gsemb)

    def wait_gather(bb):
        evb, rowsb, gsemb = bb[0], bb[2], bb[4]
        pltpu.make_async_copy(ego_hbm.at[evb.at[0]], rowsb, gsemb).wait()

    def issue_scat(bb):
        evb, rowsb, ssemb = bb[0], bb[2], bb[5]
        pltpu.async_copy(rowsb, acc.at[evb.at[1]], ssemb, add=True)

    def wait_scat(bb):
        evb, rowsb, ssemb = bb[0], bb[2], bb[5]
        pltpu.make_async_copy(rowsb, acc.at[evb.at[1]], ssemb).wait()

    def scale(bb):
        avb, rowsb = bb[1], bb[2]

        def scale_body(e16, _):
            base16 = e16 * 16
            seg = avb[pl.ds(base16, 16)]
            for u in range(16):
                e = base16 + u
                ab = seg.at[jnp.full((16,), u, jnp.int32)].get(
                    mode="promise_in_bounds")
                for j in range(D // 16):
                    rowsb[e, pl.ds(j * 16, 16)] = (
                        rowsb[e, pl.ds(j * 16, 16)] * ab)
            return 0

        lax.fori_loop(0, CHUNK // 16, scale_body, 0)

    K = len(bufs)

    issue_idx(0, bufs[0])
    issue_idx(1, bufs[1])
    issue_idx(2, bufs[2])

    zrows, zsem = bufs[4][2], bufs[4][5]

    def zero_body(e, _):
        zero = jnp.zeros((16,), jnp.float32)
        for j in range(D // 16):
            zrows[e, pl.ds(j * 16, 16)] = zero
        return 0

    lax.fori_loop(0, ZCH, zero_body, 0)

    def zero_acc_body(k, _):
        row0 = pl.multiple_of(k * ZCH, 8)
        pltpu.async_copy(zrows.at[pl.ds(0, ZCH)], acc.at[pl.ds(row0, ZCH)],
                         zsem)
        return 0

    lax.fori_loop(zlo, zhi, zero_acc_body, 0)

    wait_idx(bufs[0])
    issue_gather(bufs[0])
    wait_idx(bufs[1])
    issue_gather(bufs[1])

    def zero_drain(k, _):
        pltpu.make_async_copy(zrows.at[pl.ds(0, ZCH)],
                              acc.at[pl.ds(0, ZCH)], zsem).wait()
        return 0

    lax.fori_loop(0, nz, zero_drain, 0)
    plsc.subcore_barrier()

    def outer(gg, _):
        for b5 in range(K):
            g = K * gg + b5
            cur = bufs[b5]

            @pl.when(g < n)
            def _():
                @pl.when(g + 3 < n)
                def _():
                    nb = bufs[(b5 + 3) % K]

                    @pl.when(g >= 2)
                    def _():
                        wait_scat(nb)

                    issue_idx(g + 3, nb)

                @pl.when(g + 2 < n)
                def _():
                    gb = bufs[(b5 + 2) % K]
                    wait_idx(gb)
                    issue_gather(gb)

                wait_gather(cur)
                scale(cur)
                issue_scat(cur)

        return 0

    lax.fori_loop(0, (NCHUNK // NW + 1 + K - 1) // K, outer, 0)
    for bb in bufs:
        wait_scat(bb)
    plsc.subcore_barrier()

    def copyout_outer(kk, _):
        for b in range(K):
            j = K * kk + b

            @pl.when(j < nz)
            def _():
                rb, sb = bufs[b][2], bufs[b][5]
                row0 = pl.multiple_of((zlo + j) * ZCH, 8)

                @pl.when(kk >= 1)
                def _():
                    pltpu.make_async_copy(rb.at[pl.ds(0, ZCH)],
                                          out_hbm.at[c, pl.ds(0, ZCH)],
                                          sb).wait()

                pltpu.sync_copy(acc.at[pl.ds(row0, ZCH)],
                                rb.at[pl.ds(0, ZCH)])
                pltpu.async_copy(rb.at[pl.ds(0, ZCH)],
                                 out_hbm.at[c, pl.ds(row0, ZCH)], sb)

        return 0

    lax.fori_loop(0, (NROWCH // NS + 1 + K - 1) // K + 1, copyout_outer, 0)
    for b in range(K):
        pltpu.make_async_copy(bufs[b][2].at[pl.ds(0, ZCH)],
                              out_hbm.at[c, pl.ds(0, ZCH)],
                              bufs[b][5]).wait()


_sc_scatter = pl.kernel(
    _sc_scatter_body,
    out_type=jax.ShapeDtypeStruct((NC, N, D), jnp.float32),
    mesh=_mesh,
    scratch_types=(
        [pltpu.VMEM_SHARED((N, D), jnp.float32)]
        + [pltpu.VMEM((2, CHUNK), jnp.int32)] * 5
        + [pltpu.VMEM((CHUNK,), jnp.float32)] * 5
        + [pltpu.VMEM((CHUNK, D), jnp.float32)] * 5
        + [pltpu.SemaphoreType.DMA] * 15
    ),
)


def _sc_gather_body(t0, t1, t2, uids, pids, nids, out_hbm, idxv, rowsv, sem):
    c = lax.axis_index("c")
    s = lax.axis_index("s")
    w = s * NC + c
    base = pl.multiple_of(w * BPW, 8)
    for q, ids_hbm in enumerate((uids, pids, nids)):
        pltpu.sync_copy(ids_hbm.at[pl.ds(base, BPW)], idxv.at[0])
        for t, tab in enumerate((t0, t1, t2)):
            pltpu.async_copy(tab.at[idxv.at[0]], rowsv, sem).wait()
            pltpu.sync_copy(rowsv, out_hbm.at[t, q, pl.ds(base, BPW)])


_sc_gather = pl.kernel(
    _sc_gather_body,
    out_type=jax.ShapeDtypeStruct((3, 3, B, D), jnp.float32),
    mesh=_mesh,
    scratch_types=[
        pltpu.VMEM((1, BPW), jnp.int32),
        pltpu.VMEM((BPW, D), jnp.float32),
        pltpu.SemaphoreType.DMA,
    ],
)


def _tc_layer_body(ego_ref, p_ref, w1_ref, b1_ref, w2_ref, b2_ref,
                   oego_ref, onorm_ref):
    ego = ego_ref[...]
    agg = p_ref[0] + p_ref[1]
    h1 = jnp.dot(ego + agg, w1_ref[...],
                 preferred_element_type=jnp.float32) + b1_ref[...]
    h1 = jnp.where(h1 >= 0, h1, 0.01 * h1)
    h2 = jnp.dot(ego * agg, w2_ref[...],
                 preferred_element_type=jnp.float32) + b2_ref[...]
    h2 = jnp.where(h2 >= 0, h2, 0.01 * h2)
    newego = h1 + h2
    oego_ref[...] = newego
    nrm = jnp.sqrt(jnp.sum(newego * newego, axis=1, keepdims=True)) + 1e-12
    onorm_ref[...] = newego / nrm


_TC_R = 1000


def _tc_layer(ego, parts, w1, b1, w2, b2):
    grid = (N // _TC_R,)
    return pl.pallas_call(
        _tc_layer_body,
        grid=grid,
        in_specs=[
            pl.BlockSpec((_TC_R, D), lambda i: (i, 0)),
            pl.BlockSpec((NC, _TC_R, D), lambda i: (0, i, 0)),
            pl.BlockSpec((D, D), lambda i: (0, 0)),
            pl.BlockSpec((1, D), lambda i: (0, 0)),
            pl.BlockSpec((D, D), lambda i: (0, 0)),
            pl.BlockSpec((1, D), lambda i: (0, 0)),
        ],
        out_specs=[
            pl.BlockSpec((_TC_R, D), lambda i: (i, 0)),
            pl.BlockSpec((_TC_R, D), lambda i: (i, 0)),
        ],
        out_shape=[
            jax.ShapeDtypeStruct((N, D), jnp.float32),
            jax.ShapeDtypeStruct((N, D), jnp.float32),
        ],
    )(ego, parts, w1, b1.reshape(1, D), w2, b2.reshape(1, D))


def _tc_loss_body(g_ref, base_ref, reg_ref):
    sp = jnp.zeros((B, 1), jnp.float32)
    sn = jnp.zeros((B, 1), jnp.float32)
    l2 = jnp.float32(0.0)
    for t in range(3):
        u = g_ref[t, 0]
        p = g_ref[t, 1]
        n = g_ref[t, 2]
        sp = sp + jnp.sum(u * p, axis=1, keepdims=True)
        sn = sn + jnp.sum(u * n, axis=1, keepdims=True)
        l2 = l2 + jnp.sum(u * u) + jnp.sum(p * p) + jnp.sum(n * n)
    x = -(sp - sn)
    softplus = jnp.maximum(x, 0.0) + jnp.log1p(jnp.exp(-jnp.abs(x)))
    base_ref[...] = jnp.sum(softplus).reshape(1, 1)
    reg_ref[...] = (jnp.float32(REG) * 0.5 * l2).reshape(1, 1)


def _tc_loss(gathered):
    return pl.pallas_call(
        _tc_loss_body,
        out_shape=[
            jax.ShapeDtypeStruct((1, 1), jnp.float32),
            jax.ShapeDtypeStruct((1, 1), jnp.float32),
        ],
    )(gathered)


def kernel(entity_table, attention, w1_0, b1_0, w2_0, b2_0, w1_1, b1_1,
           w2_1, b2_1, edge_index, userids, itemids_pos, itemids_neg):
    src = edge_index[0]
    dst = edge_index[1]
    att = attention.reshape(E)

    parts0 = _sc_scatter(entity_table, src, dst, att)
    ego1, norm1 = _tc_layer(entity_table, parts0, w1_0, b1_0, w2_0, b2_0)
    parts1 = _sc_scatter(ego1, src, dst, att)
    _, norm2 = _tc_layer(ego1, parts1, w1_1, b1_1, w2_1, b2_1)

    gathered = _sc_gather(entity_table, norm1, norm2, userids, itemids_pos,
                          itemids_neg)
    base, reg = _tc_loss(gathered)
    return (base.reshape(()), reg.reshape(()))


# speedup vs baseline: 4.6011x; 1.0079x over previous
import functools

import jax
import jax.numpy as jnp
from jax import lax
from jax.experimental import pallas as pl
from jax.experimental.pallas import tpu as pltpu
from jax.experimental.pallas import tpu_sc as plsc

N = 10000
E = 320000
D = 128
B = 1024
REG = 1e-05

NC = 2
NS = 16
NW = NC * NS
CHUNK = 64
NCHUNK = E // CHUNK
ZCH = 40
NROWCH = N // ZCH
BPW = B // NW

_mesh = plsc.VectorSubcoreMesh(core_axis_name="c", subcore_axis_name="s")


def _sc_scatter_body(ego_hbm, src_hbm, dst_hbm, att_hbm, out_hbm,
                     acc,
                     ev0, ev1, ev2, ev3, ev4,
                     av0, av1, av2, av3, av4,
                     rows0, rows1, rows2, rows3, rows4,
                     isem0, isem1, isem2, isem3, isem4,
                     gsem0, gsem1, gsem2, gsem3, gsem4,
                     ssem0, ssem1, ssem2, ssem3, ssem4):
    c = lax.axis_index("c")
    s = lax.axis_index("s")
    w = s * NC + c
    n = (NCHUNK - w + NW - 1) // NW
    zlo = (s * NROWCH) // NS
    zhi = ((s + 1) * NROWCH) // NS

    nz = zhi - zlo

    bufs = ((ev0, av0, rows0, isem0, gsem0, ssem0),
            (ev1, av1, rows1, isem1, gsem1, ssem1),
            (ev2, av2, rows2, isem2, gsem2, ssem2),
            (ev3, av3, rows3, isem3, gsem3, ssem3),
            (ev4, av4, rows4, isem4, gsem4, ssem4))

    def issue_idx(g, bb):
        evb, avb, isemb = bb[0], bb[1], bb[3]
        cid = w + NW * g
        base = pl.multiple_of(cid * CHUNK, 8)
        pltpu.async_copy(src_hbm.at[pl.ds(base, CHUNK)], evb.at[0], isemb)
        pltpu.async_copy(dst_hbm.at[pl.ds(base, CHUNK)], evb.at[1], isemb)
        pltpu.async_copy(att_hbm.at[pl.ds(base, CHUNK)], avb, isemb)

    def wait_idx(bb):
        evb, avb, isemb = bb[0], bb[1], bb[3]
        pltpu.make_async_copy(src_hbm.at[pl.ds(0, CHUNK)], evb.at[0],
                              isemb).wait()
        pltpu.make_async_copy(dst_hbm.at[pl.ds(0, CHUNK)], evb.at[1],
                              isemb).wait()
        pltpu.make_async_copy(att_hbm.at[pl.ds(0, CHUNK)], avb,
                              isemb).wait()

    def issue_gather(bb):
        evb, rowsb, gsemb = bb[0], bb[2], bb[4]
        pltpu.async_copy(ego_hbm.at[evb.at[0]], rowsb, gsemb)

    def wait_gather(bb):
        evb, rowsb, gsemb = bb[0], bb[2], bb[4]
        pltpu.make_async_copy(ego_hbm.at[evb.at[0]], rowsb, gsemb).wait()

    def issue_scat(bb):
        evb, rowsb, ssemb = bb[0], bb[2], bb[5]
        pltpu.async_copy(rowsb, acc.at[evb.at[1]], ssemb, add=True)

    def wait_scat(bb):
        evb, rowsb, ssemb = bb[0], bb[2], bb[5]
        pltpu.make_async_copy(rowsb, acc.at[evb.at[1]], ssemb).wait()

    def scale(bb):
        avb, rowsb = bb[1], bb[2]

        def scale_body(e16, _):
            base16 = e16 * 16
            seg = avb[pl.ds(base16, 16)]
            for u in range(16):
                e = base16 + u
                ab = seg.at[jnp.full((16,), u, jnp.int32)].get(
                    mode="promise_in_bounds")
                for j in range(D // 16):
                    rowsb[e, pl.ds(j * 16, 16)] = (
                        rowsb[e, pl.ds(j * 16, 16)] * ab)
            return 0

        lax.fori_loop(0, CHUNK // 16, scale_body, 0)

    K = len(bufs)

    issue_idx(0, bufs[0])
    issue_idx(1, bufs[1])
    issue_idx(2, bufs[2])

    zrows, zsem = bufs[4][2], bufs[4][5]

    def zero_body(e, _):
        zero = jnp.zeros((16,), jnp.float32)
        for j in range(D // 16):
            zrows[e, pl.ds(j * 16, 16)] = zero
        return 0

    lax.fori_loop(0, ZCH, zero_body, 0)

    def zero_acc_body(k, _):
        row0 = pl.multiple_of(k * ZCH, 8)
        pltpu.async_copy(zrows.at[pl.ds(0, ZCH)], acc.at[pl.ds(row0, ZCH)],
                         zsem)
        return 0

    lax.fori_loop(zlo, zhi, zero_acc_body, 0)

    wait_idx(bufs[0])
    issue_gather(bufs[0])
    wait_idx(bufs[1])
    issue_gather(bufs[1])

    def zero_drain(k, _):
        pltpu.make_async_copy(zrows.at[pl.ds(0, ZCH)],
                              acc.at[pl.ds(0, ZCH)], zsem).wait()
        return 0

    lax.fori_loop(0, nz, zero_drain, 0)
    plsc.subcore_barrier()

    def outer(gg, _):
        for b5 in range(K):
            g = K * gg + b5
            cur = bufs[b5]

            @pl.when(g < n)
            def _():
                @pl.when(g + 3 < n)
                def _():
                    nb = bufs[(b5 + 3) % K]

                    @pl.when(g >= 2)
                    def _():
                        wait_scat(nb)

                    issue_idx(g + 3, nb)

                @pl.when(g + 2 < n)
                def _():
                    gb = bufs[(b5 + 2) % K]
                    wait_idx(gb)
                    issue_gather(gb)

                wait_gather(cur)
                scale(cur)
                issue_scat(cur)

        return 0

    lax.fori_loop(0, (NCHUNK // NW + 1 + K - 1) // K, outer, 0)
    for bb in bufs:
        wait_scat(bb)
    plsc.subcore_barrier()

    def copyout_outer(kk, _):
        for b in range(K):
            j = K * kk + b

            @pl.when(j < nz)
            def _():
                rb, sb = bufs[b][2], bufs[b][5]
                row0 = pl.multiple_of((zlo + j) * ZCH, 8)

                @pl.when(kk >= 1)
                def _():
                    pltpu.make_async_copy(rb.at[pl.ds(0, ZCH)],
                                          out_hbm.at[c, pl.ds(0, ZCH)],
                                          sb).wait()

                pltpu.sync_copy(acc.at[pl.ds(row0, ZCH)],
                                rb.at[pl.ds(0, ZCH)])
                pltpu.async_copy(rb.at[pl.ds(0, ZCH)],
                                 out_hbm.at[c, pl.ds(row0, ZCH)], sb)

        return 0

    lax.fori_loop(0, (NROWCH // NS + 1 + K - 1) // K + 1, copyout_outer, 0)
    for b in range(K):
        pltpu.make_async_copy(bufs[b][2].at[pl.ds(0, ZCH)],
                              out_hbm.at[c, pl.ds(0, ZCH)],
                              bufs[b][5]).wait()


_sc_scatter = pl.kernel(
    _sc_scatter_body,
    out_type=jax.ShapeDtypeStruct((NC, N, D), jnp.float32),
    mesh=_mesh,
    scratch_types=(
        [pltpu.VMEM_SHARED((N, D), jnp.float32)]
        + [pltpu.VMEM((2, CHUNK), jnp.int32)] * 5
        + [pltpu.VMEM((CHUNK,), jnp.float32)] * 5
        + [pltpu.VMEM((CHUNK, D), jnp.float32)] * 5
        + [pltpu.SemaphoreType.DMA] * 15
    ),
)


def _sc_gather_body(t0, t1, t2, uids, pids, nids, out_hbm, idxv,
                    r0, r1, gsem0, gsem1, osem0, osem1):
    c = lax.axis_index("c")
    s = lax.axis_index("s")
    w = s * NC + c
    base = pl.multiple_of(w * BPW, 8)
    pltpu.sync_copy(uids.at[pl.ds(base, BPW)], idxv.at[0])
    pltpu.sync_copy(pids.at[pl.ds(base, BPW)], idxv.at[1])
    pltpu.sync_copy(nids.at[pl.ds(base, BPW)], idxv.at[2])
    tabs = (t0, t1, t2)
    gbufs = ((r0, gsem0, osem0), (r1, gsem1, osem1))
    jobs = [(q, t) for q in range(3) for t in range(3)]
    pltpu.async_copy(tabs[0].at[idxv.at[0]], r0, gsem0)
    for j, (q, t) in enumerate(jobs):
        rb, gs, os_ = gbufs[j % 2]
        pltpu.make_async_copy(tabs[t].at[idxv.at[q]], rb, gs).wait()
        if j + 1 < len(jobs):
            nrb, ngs, nos = gbufs[(j + 1) % 2]
            if j >= 1:
                pltpu.make_async_copy(nrb, out_hbm.at[0, 0, pl.ds(0, BPW)],
                                      nos).wait()
            qq, tt = jobs[j + 1]
            pltpu.async_copy(tabs[tt].at[idxv.at[qq]], nrb, ngs)
        pltpu.async_copy(rb, out_hbm.at[t, q, pl.ds(base, BPW)], os_)
    pltpu.make_async_copy(r1, out_hbm.at[0, 0, pl.ds(0, BPW)], osem1).wait()
    pltpu.make_async_copy(r0, out_hbm.at[0, 0, pl.ds(0, BPW)], osem0).wait()


_sc_gather = pl.kernel(
    _sc_gather_body,
    out_type=jax.ShapeDtypeStruct((3, 3, B, D), jnp.float32),
    mesh=_mesh,
    scratch_types=[
        pltpu.VMEM((3, BPW), jnp.int32),
        pltpu.VMEM((BPW, D), jnp.float32),
        pltpu.VMEM((BPW, D), jnp.float32),
        pltpu.SemaphoreType.DMA,
        pltpu.SemaphoreType.DMA,
        pltpu.SemaphoreType.DMA,
        pltpu.SemaphoreType.DMA,
    ],
)


def _tc_layer_body(ego_ref, p_ref, w1_ref, b1_ref, w2_ref, b2_ref,
                   oego_ref, onorm_ref):
    ego = ego_ref[...]
    agg = p_ref[0] + p_ref[1]
    h1 = jnp.dot(ego + agg, w1_ref[...],
                 preferred_element_type=jnp.float32) + b1_ref[...]
    h1 = jnp.where(h1 >= 0, h1, 0.01 * h1)
    h2 = jnp.dot(ego * agg, w2_ref[...],
                 preferred_element_type=jnp.float32) + b2_ref[...]
    h2 = jnp.where(h2 >= 0, h2, 0.01 * h2)
    newego = h1 + h2
    oego_ref[...] = newego
    nrm = jnp.sqrt(jnp.sum(newego * newego, axis=1, keepdims=True)) + 1e-12
    onorm_ref[...] = newego / nrm


def _tc_layer2_body(ego_ref, p_ref, w1_ref, b1_ref, w2_ref, b2_ref,
                    onorm_ref):
    ego = ego_ref[...]
    agg = p_ref[0] + p_ref[1]
    h1 = jnp.dot(ego + agg, w1_ref[...],
                 preferred_element_type=jnp.float32) + b1_ref[...]
    h1 = jnp.where(h1 >= 0, h1, 0.01 * h1)
    h2 = jnp.dot(ego * agg, w2_ref[...],
                 preferred_element_type=jnp.float32) + b2_ref[...]
    h2 = jnp.where(h2 >= 0, h2, 0.01 * h2)
    newego = h1 + h2
    nrm = jnp.sqrt(jnp.sum(newego * newego, axis=1, keepdims=True)) + 1e-12
    onorm_ref[...] = newego / nrm


_TC_R = 1000


def _tc_layer2(ego, parts, w1, b1, w2, b2):
    grid = (N // _TC_R,)
    return pl.pallas_call(
        _tc_layer2_body,
        grid=grid,
        in_specs=[
            pl.BlockSpec((_TC_R, D), lambda i: (i, 0)),
            pl.BlockSpec((NC, _TC_R, D), lambda i: (0, i, 0)),
            pl.BlockSpec((D, D), lambda i: (0, 0)),
            pl.BlockSpec((1, D), lambda i: (0, 0)),
            pl.BlockSpec((D, D), lambda i: (0, 0)),
            pl.BlockSpec((1, D), lambda i: (0, 0)),
        ],
        out_specs=pl.BlockSpec((_TC_R, D), lambda i: (i, 0)),
        out_shape=jax.ShapeDtypeStruct((N, D), jnp.float32),
    )(ego, parts, w1, b1.reshape(1, D), w2, b2.reshape(1, D))


def _tc_layer(ego, parts, w1, b1, w2, b2):
    grid = (N // _TC_R,)
    return pl.pallas_call(
        _tc_layer_body,
        grid=grid,
        in_specs=[
            pl.BlockSpec((_TC_R, D), lambda i: (i, 0)),
            pl.BlockSpec((NC, _TC_R, D), lambda i: (0, i, 0)),
            pl.BlockSpec((D, D), lambda i: (0, 0)),
            pl.BlockSpec((1, D), lambda i: (0, 0)),
            pl.BlockSpec((D, D), lambda i: (0, 0)),
            pl.BlockSpec((1, D), lambda i: (0, 0)),
        ],
        out_specs=[
            pl.BlockSpec((_TC_R, D), lambda i: (i, 0)),
            pl.BlockSpec((_TC_R, D), lambda i: (i, 0)),
        ],
        out_shape=[
            jax.ShapeDtypeStruct((N, D), jnp.float32),
            jax.ShapeDtypeStruct((N, D), jnp.float32),
        ],
    )(ego, parts, w1, b1.reshape(1, D), w2, b2.reshape(1, D))


def _tc_loss_body(g_ref, base_ref, reg_ref):
    sp = jnp.zeros((B, 1), jnp.float32)
    sn = jnp.zeros((B, 1), jnp.float32)
    l2 = jnp.float32(0.0)
    for t in range(3):
        u = g_ref[t, 0]
        p = g_ref[t, 1]
        n = g_ref[t, 2]
        sp = sp + jnp.sum(u * p, axis=1, keepdims=True)
        sn = sn + jnp.sum(u * n, axis=1, keepdims=True)
        l2 = l2 + jnp.sum(u * u) + jnp.sum(p * p) + jnp.sum(n * n)
    x = -(sp - sn)
    softplus = jnp.maximum(x, 0.0) + jnp.log1p(jnp.exp(-jnp.abs(x)))
    base_ref[...] = jnp.sum(softplus).reshape(1, 1)
    reg_ref[...] = (jnp.float32(REG) * 0.5 * l2).reshape(1, 1)


def _tc_loss(gathered):
    return pl.pallas_call(
        _tc_loss_body,
        out_shape=[
            jax.ShapeDtypeStruct((1, 1), jnp.float32),
            jax.ShapeDtypeStruct((1, 1), jnp.float32),
        ],
    )(gathered)


def kernel(entity_table, attention, w1_0, b1_0, w2_0, b2_0, w1_1, b1_1,
           w2_1, b2_1, edge_index, userids, itemids_pos, itemids_neg):
    src = edge_index[0]
    dst = edge_index[1]
    att = attention.reshape(E)

    parts0 = _sc_scatter(entity_table, src, dst, att)
    ego1, norm1 = _tc_layer(entity_table, parts0, w1_0, b1_0, w2_0, b2_0)
    parts1 = _sc_scatter(ego1, src, dst, att)
    norm2 = _tc_layer2(ego1, parts1, w1_1, b1_1, w2_1, b2_1)

    gathered = _sc_gather(entity_table, norm1, norm2, userids, itemids_pos,
                          itemids_neg)
    base, reg = _tc_loss(gathered)
    return (base.reshape(()), reg.reshape(()))


# speedup vs baseline: 4.6593x; 1.0126x over previous
import functools

import jax
import jax.numpy as jnp
from jax import lax
from jax.experimental import pallas as pl
from jax.experimental.pallas import tpu as pltpu
from jax.experimental.pallas import tpu_sc as plsc

N = 10000
E = 320000
D = 128
B = 1024
REG = 1e-05

NC = 2
NS = 16
NW = NC * NS
CHUNK = 64
NCHUNK = E // CHUNK
ZCH = 40
NROWCH = N // ZCH
BPW = B // NW

_mesh = plsc.VectorSubcoreMesh(core_axis_name="c", subcore_axis_name="s")


def _sc_scatter_body(ego_hbm, src_hbm, dst_hbm, att_hbm, out_hbm,
                     acc,
                     ev0, ev1, ev2, ev3, ev4,
                     av0, av1, av2, av3, av4,
                     rows0, rows1, rows2, rows3, rows4,
                     isem0, isem1, isem2, isem3, isem4,
                     gsem0, gsem1, gsem2, gsem3, gsem4,
                     ssem0, ssem1, ssem2, ssem3, ssem4):
    c = lax.axis_index("c")
    s = lax.axis_index("s")
    w = s * NC + c
    n = (NCHUNK - w + NW - 1) // NW
    zlo = (s * NROWCH) // NS
    zhi = ((s + 1) * NROWCH) // NS

    nz = zhi - zlo

    bufs = ((ev0, av0, rows0, isem0, gsem0, ssem0),
            (ev1, av1, rows1, isem1, gsem1, ssem1),
            (ev2, av2, rows2, isem2, gsem2, ssem2),
            (ev3, av3, rows3, isem3, gsem3, ssem3),
            (ev4, av4, rows4, isem4, gsem4, ssem4))

    def issue_idx(g, bb):
        evb, avb, isemb = bb[0], bb[1], bb[3]
        cid = w + NW * g
        base = pl.multiple_of(cid * CHUNK, 8)
        pltpu.async_copy(src_hbm.at[pl.ds(base, CHUNK)], evb.at[0], isemb)
        pltpu.async_copy(dst_hbm.at[pl.ds(base, CHUNK)], evb.at[1], isemb)
        pltpu.async_copy(att_hbm.at[pl.ds(base, CHUNK)], avb, isemb)

    def wait_idx(bb):
        evb, avb, isemb = bb[0], bb[1], bb[3]
        pltpu.make_async_copy(src_hbm.at[pl.ds(0, CHUNK)], evb.at[0],
                              isemb).wait()
        pltpu.make_async_copy(dst_hbm.at[pl.ds(0, CHUNK)], evb.at[1],
                              isemb).wait()
        pltpu.make_async_copy(att_hbm.at[pl.ds(0, CHUNK)], avb,
                              isemb).wait()

    def issue_gather(bb):
        evb, rowsb, gsemb = bb[0], bb[2], bb[4]
        pltpu.async_copy(ego_hbm.at[evb.at[0]], rowsb, gsemb)

    def wait_gather(bb):
        evb, rowsb, gsemb = bb[0], bb[2], bb[4]
        pltpu.make_async_copy(ego_hbm.at[evb.at[0]], rowsb, gsemb).wait()

    def issue_scat(bb):
        evb, rowsb, ssemb = bb[0], bb[2], bb[5]
        pltpu.async_copy(rowsb, acc.at[evb.at[1]], ssemb, add=True)

    def wait_scat(bb):
        evb, rowsb, ssemb = bb[0], bb[2], bb[5]
        pltpu.make_async_copy(rowsb, acc.at[evb.at[1]], ssemb).wait()

    def scale(bb):
        avb, rowsb = bb[1], bb[2]

        def scale_body(e16, _):
            base16 = e16 * 16
            seg = avb[pl.ds(base16, 16)]
            for u in range(16):
                e = base16 + u
                ab = seg.at[jnp.full((16,), u, jnp.int32)].get(
                    mode="promise_in_bounds")
                for j in range(D // 16):
                    rowsb[e, pl.ds(j * 16, 16)] = (
                        rowsb[e, pl.ds(j * 16, 16)] * ab)
            return 0

        lax.fori_loop(0, CHUNK // 16, scale_body, 0)

    K = len(bufs)

    issue_idx(0, bufs[0])
    issue_idx(1, bufs[1])
    issue_idx(2, bufs[2])

    zrows, zsem = bufs[4][2], bufs[4][5]

    def zero_body(e, _):
        zero = jnp.zeros((16,), jnp.float32)
        for j in range(D // 16):
            zrows[e, pl.ds(j * 16, 16)] = zero
        return 0

    lax.fori_loop(0, ZCH, zero_body, 0)

    def zero_acc_body(k, _):
        row0 = pl.multiple_of(k * ZCH, 8)
        pltpu.async_copy(zrows.at[pl.ds(0, ZCH)], acc.at[pl.ds(row0, ZCH)],
                         zsem)
        return 0

    lax.fori_loop(zlo, zhi, zero_acc_body, 0)

    wait_idx(bufs[0])
    issue_gather(bufs[0])
    wait_idx(bufs[1])
    issue_gather(bufs[1])

    def zero_drain(k, _):
        pltpu.make_async_copy(zrows.at[pl.ds(0, ZCH)],
                              acc.at[pl.ds(0, ZCH)], zsem).wait()
        return 0

    lax.fori_loop(0, nz, zero_drain, 0)
    plsc.subcore_barrier()

    def outer(gg, _):
        for b5 in range(K):
            g = K * gg + b5
            cur = bufs[b5]

            @pl.when(g < n)
            def _():
                @pl.when(g + 3 < n)
                def _():
                    nb = bufs[(b5 + 3) % K]

                    @pl.when(g >= 2)
                    def _():
                        wait_scat(nb)

                    issue_idx(g + 3, nb)

                @pl.when(g + 2 < n)
                def _():
                    gb = bufs[(b5 + 2) % K]
                    wait_idx(gb)
                    issue_gather(gb)

                wait_gather(cur)
                scale(cur)
                issue_scat(cur)

        return 0

    lax.fori_loop(0, (NCHUNK // NW + 1 + K - 1) // K, outer, 0)
    for bb in bufs:
        wait_scat(bb)
    plsc.subcore_barrier()

    def copyout_outer(kk, _):
        for b in range(K):
            j = K * kk + b

            @pl.when(j < nz)
            def _():
                rb, sb = bufs[b][2], bufs[b][5]
                row0 = pl.multiple_of((zlo + j) * ZCH, 8)

                @pl.when(kk >= 1)
                def _():
                    pltpu.make_async_copy(rb.at[pl.ds(0, ZCH)],
                                          out_hbm.at[c, pl.ds(0, ZCH)],
                                          sb).wait()

                pltpu.sync_copy(acc.at[pl.ds(row0, ZCH)],
                                rb.at[pl.ds(0, ZCH)])
                pltpu.async_copy(rb.at[pl.ds(0, ZCH)],
                                 out_hbm.at[c, pl.ds(row0, ZCH)], sb)

        return 0

    lax.fori_loop(0, (NROWCH // NS + 1 + K - 1) // K + 1, copyout_outer, 0)
    for b in range(K):
        pltpu.make_async_copy(bufs[b][2].at[pl.ds(0, ZCH)],
                              out_hbm.at[c, pl.ds(0, ZCH)],
                              bufs[b][5]).wait()


_sc_scatter = pl.kernel(
    _sc_scatter_body,
    out_type=jax.ShapeDtypeStruct((NC, N, D), jnp.float32),
    mesh=_mesh,
    scratch_types=(
        [pltpu.VMEM_SHARED((N, D), jnp.float32)]
        + [pltpu.VMEM((2, CHUNK), jnp.int32)] * 5
        + [pltpu.VMEM((CHUNK,), jnp.float32)] * 5
        + [pltpu.VMEM((CHUNK, D), jnp.float32)] * 5
        + [pltpu.SemaphoreType.DMA] * 15
    ),
)


def _sc_gather_body(t0, t1, t2, uids, pids, nids, out_hbm, idxv,
                    r0, r1, gsem0, gsem1, osem0, osem1):
    c = lax.axis_index("c")
    s = lax.axis_index("s")
    w = s * NC + c
    base = pl.multiple_of(w * BPW, 8)
    pltpu.sync_copy(uids.at[pl.ds(base, BPW)], idxv.at[0])
    pltpu.sync_copy(pids.at[pl.ds(base, BPW)], idxv.at[1])
    pltpu.sync_copy(nids.at[pl.ds(base, BPW)], idxv.at[2])
    tabs = (t0, t1, t2)
    gbufs = ((r0, gsem0, osem0), (r1, gsem1, osem1))
    jobs = [(q, t) for q in range(3) for t in range(3)]
    pltpu.async_copy(tabs[0].at[idxv.at[0]], r0, gsem0)
    for j, (q, t) in enumerate(jobs):
        rb, gs, os_ = gbufs[j % 2]
        pltpu.make_async_copy(tabs[t].at[idxv.at[q]], rb, gs).wait()
        if j + 1 < len(jobs):
            nrb, ngs, nos = gbufs[(j + 1) % 2]
            if j >= 1:
                pltpu.make_async_copy(nrb, out_hbm.at[0, 0, pl.ds(0, BPW)],
                                      nos).wait()
            qq, tt = jobs[j + 1]
            pltpu.async_copy(tabs[tt].at[idxv.at[qq]], nrb, ngs)
        pltpu.async_copy(rb, out_hbm.at[t, q, pl.ds(base, BPW)], os_)
    pltpu.make_async_copy(r1, out_hbm.at[0, 0, pl.ds(0, BPW)], osem1).wait()
    pltpu.make_async_copy(r0, out_hbm.at[0, 0, pl.ds(0, BPW)], osem0).wait()


_sc_gather = pl.kernel(
    _sc_gather_body,
    out_type=jax.ShapeDtypeStruct((3, 3, B, D), jnp.float32),
    mesh=_mesh,
    scratch_types=[
        pltpu.VMEM((3, BPW), jnp.int32),
        pltpu.VMEM((BPW, D), jnp.float32),
        pltpu.VMEM((BPW, D), jnp.float32),
        pltpu.SemaphoreType.DMA,
        pltpu.SemaphoreType.DMA,
        pltpu.SemaphoreType.DMA,
        pltpu.SemaphoreType.DMA,
    ],
)


def _tc_layer_body(ego_ref, p_ref, w1_ref, b1_ref, w2_ref, b2_ref,
                   oego_ref, onorm_ref):
    ego = ego_ref[...]
    agg = p_ref[0] + p_ref[1]
    h1 = jnp.dot(ego + agg, w1_ref[...],
                 preferred_element_type=jnp.float32) + b1_ref[...]
    h1 = jnp.where(h1 >= 0, h1, 0.01 * h1)
    h2 = jnp.dot(ego * agg, w2_ref[...],
                 preferred_element_type=jnp.float32) + b2_ref[...]
    h2 = jnp.where(h2 >= 0, h2, 0.01 * h2)
    newego = h1 + h2
    oego_ref[...] = newego
    nrm = jnp.sqrt(jnp.sum(newego * newego, axis=1, keepdims=True)) + 1e-12
    onorm_ref[...] = newego / nrm


def _tc_layer2_body(ego_ref, p_ref, w1_ref, b1_ref, w2_ref, b2_ref,
                    onorm_ref):
    ego = ego_ref[...]
    agg = p_ref[0] + p_ref[1]
    h1 = jnp.dot(ego + agg, w1_ref[...],
                 preferred_element_type=jnp.float32) + b1_ref[...]
    h1 = jnp.where(h1 >= 0, h1, 0.01 * h1)
    h2 = jnp.dot(ego * agg, w2_ref[...],
                 preferred_element_type=jnp.float32) + b2_ref[...]
    h2 = jnp.where(h2 >= 0, h2, 0.01 * h2)
    newego = h1 + h2
    nrm = jnp.sqrt(jnp.sum(newego * newego, axis=1, keepdims=True)) + 1e-12
    onorm_ref[...] = newego / nrm


_TC_R = 2000


def _tc_layer2(ego, parts, w1, b1, w2, b2):
    grid = (N // _TC_R,)
    return pl.pallas_call(
        _tc_layer2_body,
        grid=grid,
        in_specs=[
            pl.BlockSpec((_TC_R, D), lambda i: (i, 0)),
            pl.BlockSpec((NC, _TC_R, D), lambda i: (0, i, 0)),
            pl.BlockSpec((D, D), lambda i: (0, 0)),
            pl.BlockSpec((1, D), lambda i: (0, 0)),
            pl.BlockSpec((D, D), lambda i: (0, 0)),
            pl.BlockSpec((1, D), lambda i: (0, 0)),
        ],
        out_specs=pl.BlockSpec((_TC_R, D), lambda i: (i, 0)),
        out_shape=jax.ShapeDtypeStruct((N, D), jnp.float32),
    )(ego, parts, w1, b1.reshape(1, D), w2, b2.reshape(1, D))


def _tc_layer(ego, parts, w1, b1, w2, b2):
    grid = (N // _TC_R,)
    return pl.pallas_call(
        _tc_layer_body,
        grid=grid,
        in_specs=[
            pl.BlockSpec((_TC_R, D), lambda i: (i, 0)),
            pl.BlockSpec((NC, _TC_R, D), lambda i: (0, i, 0)),
            pl.BlockSpec((D, D), lambda i: (0, 0)),
            pl.BlockSpec((1, D), lambda i: (0, 0)),
            pl.BlockSpec((D, D), lambda i: (0, 0)),
            pl.BlockSpec((1, D), lambda i: (0, 0)),
        ],
        out_specs=[
            pl.BlockSpec((_TC_R, D), lambda i: (i, 0)),
            pl.BlockSpec((_TC_R, D), lambda i: (i, 0)),
        ],
        out_shape=[
            jax.ShapeDtypeStruct((N, D), jnp.float32),
            jax.ShapeDtypeStruct((N, D), jnp.float32),
        ],
    )(ego, parts, w1, b1.reshape(1, D), w2, b2.reshape(1, D))


def _tc_loss_body(g_ref, base_ref, reg_ref):
    sp = jnp.zeros((B, 1), jnp.float32)
    sn = jnp.zeros((B, 1), jnp.float32)
    l2 = jnp.float32(0.0)
    for t in range(3):
        u = g_ref[t, 0]
        p = g_ref[t, 1]
        n = g_ref[t, 2]
        sp = sp + jnp.sum(u * p, axis=1, keepdims=True)
        sn = sn + jnp.sum(u * n, axis=1, keepdims=True)
        l2 = l2 + jnp.sum(u * u) + jnp.sum(p * p) + jnp.sum(n * n)
    x = -(sp - sn)
    softplus = jnp.maximum(x, 0.0) + jnp.log1p(jnp.exp(-jnp.abs(x)))
    base_ref[...] = jnp.sum(softplus).reshape(1, 1)
    reg_ref[...] = (jnp.float32(REG) * 0.5 * l2).reshape(1, 1)


def _tc_loss(gathered):
    return pl.pallas_call(
        _tc_loss_body,
        out_shape=[
            jax.ShapeDtypeStruct((1, 1), jnp.float32),
            jax.ShapeDtypeStruct((1, 1), jnp.float32),
        ],
    )(gathered)


def kernel(entity_table, attention, w1_0, b1_0, w2_0, b2_0, w1_1, b1_1,
           w2_1, b2_1, edge_index, userids, itemids_pos, itemids_neg):
    src = edge_index[0]
    dst = edge_index[1]
    att = attention.reshape(E)

    parts0 = _sc_scatter(entity_table, src, dst, att)
    ego1, norm1 = _tc_layer(entity_table, parts0, w1_0, b1_0, w2_0, b2_0)
    parts1 = _sc_scatter(ego1, src, dst, att)
    norm2 = _tc_layer2(ego1, parts1, w1_1, b1_1, w2_1, b2_1)

    gathered = _sc_gather(entity_table, norm1, norm2, userids, itemids_pos,
                          itemids_neg)
    base, reg = _tc_loss(gathered)
    return (base.reshape(()), reg.reshape(()))
